# Initial kernel scaffold; baseline (speedup 1.0000x reference)
import jax
import jax.numpy as jnp
from jax.experimental import pallas as pl


def kernel(x, pos, batch, params):
    raise NotImplementedError("write your pallas kernel here")


# speedup vs baseline: 1.0003x; 1.0003x over previous
import jax
import jax.numpy as jnp
import numpy as np
from jax.experimental import pallas as pl

NUM_CLASSES = 40
DEC = 4
KNN_K = 16
BB = 8
LL = 2048
NN = BB * LL


def _bn(y, g, b):
    m = jnp.mean(y, axis=0)
    v = jnp.mean((y - m) ** 2, axis=0)
    return (y - m) / jnp.sqrt(v + 1e-6) * g + b


def _lrelu(y):
    return jnp.where(y >= 0, y, 0.2 * y)


def _shared(p, y, act=True):
    y = y @ p["W"] + p["b"]
    y = _bn(y, p["gamma"], p["beta"])
    if act:
        y = _lrelu(y)
    return y


def _knn(pos, b, l, k):
    p = pos.reshape(b, l, 3)
    d = jnp.sum((p[:, :, None, :] - p[:, None, :, :]) ** 2, axis=-1)
    idx = jax.lax.top_k(-d, k)[1]
    src = (idx + (jnp.arange(b) * l)[:, None, None]).reshape(-1)
    dst = jnp.repeat(jnp.arange(b * l), k)
    return src, dst


def _lfa(p, x, pos, src, dst, n):
    pos_i = pos[dst]
    pos_j = pos[src]
    diff = pos_j - pos_i
    dist = jnp.sqrt(jnp.sum(diff * diff, axis=1, keepdims=True))
    rel = jnp.concatenate([pos_i, pos_j, diff, dist], axis=1)
    lse = _shared(p["enc"], rel)
    local = jnp.concatenate([x[src], lse], axis=1)
    att = local @ p["att"]["W"]
    m = jax.ops.segment_max(att, dst, num_segments=n)
    e = jnp.exp(att - m[dst])
    s = jax.ops.segment_sum(e, dst, num_segments=n)
    scores = e / (s[dst] + 1e-16)
    out = jax.ops.segment_sum(scores * local, dst, num_segments=n)
    return _shared(p["post"], out)


def _block(p, x, pos, b, l, dec):
    n = b * l
    src, dst = _knn(pos, b, l, KNN_K)
    sc = _shared(p["shortcut"], x, act=False)
    h = _shared(p["mlp1"], x)
    h = _lfa(p["lfa1"], h, pos, src, dst, n)
    h = _lfa(p["lfa2"], h, pos, src, dst, n)
    h = _shared(p["mlp2"], h, act=False)
    h = _lrelu(h + sc)
    idx = jnp.arange(0, n, dec)
    return h[idx], pos[idx]


def _head_kernel(h_ref, w1, b1, g1, be1, we1, be_1, ge1, bee1, we2, be_2, o_ref):
    h = h_ref[...]
    y = jnp.dot(h, w1[...], preferred_element_type=jnp.float32) + b1[...]
    m = jnp.mean(y, axis=0)
    v = jnp.mean((y - m) ** 2, axis=0)
    y = (y - m) / jnp.sqrt(v + 1e-6) * g1[...] + be1[...]
    y = jnp.where(y >= 0, y, 0.2 * y)
    g = jnp.max(y.reshape(BB, -1, y.shape[-1]), axis=1)
    o = jnp.dot(g, we1[...], preferred_element_type=jnp.float32) + be_1[...]
    m2 = jnp.mean(o, axis=0)
    v2 = jnp.mean((o - m2) ** 2, axis=0)
    o = (o - m2) / jnp.sqrt(v2 + 1e-6) * ge1[...] + bee1[...]
    o = jnp.where(o >= 0, o, 0.2 * o)
    logits = jnp.dot(o, we2[...], preferred_element_type=jnp.float32) + be_2[...]
    lmax = jnp.max(logits, axis=-1, keepdims=True)
    s = logits - lmax
    lse = jnp.log(jnp.sum(jnp.exp(s), axis=-1, keepdims=True))
    o_ref[...] = s - lse


def kernel(x, pos, batch, params):
    h = x @ params["fc0"]["W"] + params["fc0"]["b"]
    h, pos1 = _block(params["block1"], h, pos, BB, LL, DEC)
    h, _ = _block(params["block2"], h, pos1, BB, LL // DEC, DEC)
    p1, pe1, pe2 = params["mlp1"], params["end1"], params["end2"]
    out = pl.pallas_call(
        _head_kernel,
        out_shape=jax.ShapeDtypeStruct((BB, NUM_CLASSES), jnp.float32),
    )(h, p1["W"], p1["b"], p1["gamma"], p1["beta"],
      pe1["W"], pe1["b"], pe1["gamma"], pe1["beta"],
      pe2["W"], pe2["b"])
    return out


# speedup vs baseline: 7.9066x; 7.9045x over previous
import functools

import jax
import jax.numpy as jnp
from jax.experimental import pallas as pl
from jax.experimental.pallas import tpu as pltpu
from jax.experimental.pallas import tpu_sc as plsc

NUM_CLASSES = 40
DEC = 4
KK = 16
BB = 8
LL = 2048
NN = BB * LL

_F32 = jnp.float32


def _mm(a, b):
    return jnp.dot(a, b, preferred_element_type=jnp.float32)


def _lrelu(y):
    return jnp.where(y >= 0, y, 0.2 * y)


def _bn_apply(y, st, n, gamma, beta, act):
    d = y.shape[-1]
    s1 = st[0:1, :d]
    s2 = st[1:2, :d]
    m = s1 / n
    v = s2 / n - m * m
    out = (y - m) / jnp.sqrt(v + 1e-6) * gamma + beta
    if act:
        out = _lrelu(out)
    return out


def _stats_update(st_ref, y, step):
    @pl.when(step == 0)
    def _():
        st_ref[...] = jnp.zeros_like(st_ref)

    d = y.shape[-1]
    s1 = jnp.sum(y, axis=0, keepdims=True)
    s2 = jnp.sum(y * y, axis=0, keepdims=True)
    pad = jnp.zeros((6, d), _F32)
    st_ref[...] += jnp.concatenate([s1, s2, pad], axis=0)


def _full_spec(shape, ndim_grid=1):
    zeros = (0,) * len(shape)
    if ndim_grid == 1:
        return pl.BlockSpec(shape, lambda t: zeros)
    return pl.BlockSpec(shape, lambda *g: zeros)


def _prep1_body(x_ref, pos_ref, w0, b0, wm, bm, ws, bs,
                tab_ref, ysc_ref, st1_ref, stsc_ref):
    t = pl.program_id(0)
    h0 = _mm(x_ref[...], w0[...]) + b0[...]
    y1 = _mm(h0, wm[...]) + bm[...]
    ysc = _mm(h0, ws[...]) + bs[...]
    r = y1.shape[0]
    d = y1.shape[1]
    padw = tab_ref.shape[1] - 3 - d
    tab_ref[...] = jnp.concatenate(
        [pos_ref[...], y1, jnp.zeros((r, padw), _F32)], axis=1)
    ysc_ref[...] = ysc
    _stats_update(st1_ref, y1, t)
    _stats_update(stsc_ref, ysc, t)


def _prep2_body(h_ref, pos_ref, wm, bm, ws, bs,
                tab_ref, ysc_ref, st1_ref, stsc_ref):
    t = pl.program_id(0)
    h = h_ref[...]
    y1 = _mm(h, wm[...]) + bm[...]
    ysc = _mm(h, ws[...]) + bs[...]
    r = y1.shape[0]
    d = y1.shape[1]
    padw = tab_ref.shape[1] - 3 - d
    tab_ref[...] = jnp.concatenate(
        [pos_ref[...], y1, jnp.zeros((r, padw), _F32)], axis=1)
    ysc_ref[...] = ysc
    _stats_update(st1_ref, y1, t)
    _stats_update(stsc_ref, ysc, t)


def _run_prep1(x, pos, p_fc0, p_m, p_s, tab_w):
    n = x.shape[0]
    tile = 256
    grid = (n // tile,)
    dm = p_m["W"].shape[1]
    ds = p_s["W"].shape[1]
    return pl.pallas_call(
        _prep1_body,
        grid=grid,
        in_specs=[
            pl.BlockSpec((tile, x.shape[1]), lambda t: (t, 0)),
            pl.BlockSpec((tile, 3), lambda t: (t, 0)),
            _full_spec(p_fc0["W"].shape), _full_spec((1, p_fc0["W"].shape[1])),
            _full_spec(p_m["W"].shape), _full_spec((1, dm)),
            _full_spec(p_s["W"].shape), _full_spec((1, ds)),
        ],
        out_specs=[
            pl.BlockSpec((tile, tab_w), lambda t: (t, 0)),
            pl.BlockSpec((tile, ds), lambda t: (t, 0)),
            pl.BlockSpec((8, dm), lambda t: (0, 0)),
            pl.BlockSpec((8, ds), lambda t: (0, 0)),
        ],
        out_shape=[
            jax.ShapeDtypeStruct((n, tab_w), _F32),
            jax.ShapeDtypeStruct((n, ds), _F32),
            jax.ShapeDtypeStruct((8, dm), _F32),
            jax.ShapeDtypeStruct((8, ds), _F32),
        ],
    )(x, pos, p_fc0["W"], p_fc0["b"].reshape(1, -1),
      p_m["W"], p_m["b"].reshape(1, -1),
      p_s["W"], p_s["b"].reshape(1, -1))


def _run_prep2(h, pos, p_m, p_s, tab_w):
    n = h.shape[0]
    tile = 256
    grid = (n // tile,)
    dm = p_m["W"].shape[1]
    ds = p_s["W"].shape[1]
    return pl.pallas_call(
        _prep2_body,
        grid=grid,
        in_specs=[
            pl.BlockSpec((tile, h.shape[1]), lambda t: (t, 0)),
            pl.BlockSpec((tile, 3), lambda t: (t, 0)),
            _full_spec(p_m["W"].shape), _full_spec((1, dm)),
            _full_spec(p_s["W"].shape), _full_spec((1, ds)),
        ],
        out_specs=[
            pl.BlockSpec((tile, tab_w), lambda t: (t, 0)),
            pl.BlockSpec((tile, ds), lambda t: (t, 0)),
            pl.BlockSpec((8, dm), lambda t: (0, 0)),
            pl.BlockSpec((8, ds), lambda t: (0, 0)),
        ],
        out_shape=[
            jax.ShapeDtypeStruct((n, tab_w), _F32),
            jax.ShapeDtypeStruct((n, ds), _F32),
            jax.ShapeDtypeStruct((8, dm), _F32),
            jax.ShapeDtypeStruct((8, ds), _F32),
        ],
    )(h, pos, p_m["W"], p_m["b"].reshape(1, -1),
      p_s["W"], p_s["b"].reshape(1, -1))


_KNN_ROWS = 64


def _knn_body(l, pos_ref, posT_ref, idx_ref):
    b = pl.program_id(0)
    tile = pos_ref[0]
    d = None
    for c in range(3):
        tc = tile[:, c:c + 1]
        fc = posT_ref[0, c:c + 1, :]
        dc = (tc - fc) ** 2
        d = dc if d is None else d + dc
    iota = jax.lax.broadcasted_iota(jnp.int32, (_KNN_ROWS, l), 1)
    cols = []
    for _ in range(KK):
        m = jnp.min(d, axis=1, keepdims=True)
        cand = jnp.where(d == m, iota, jnp.int32(l))
        j = jnp.min(cand, axis=1, keepdims=True)
        cols.append(j)
        d = jnp.where(iota == j, jnp.inf, d)
    idx = jnp.concatenate(cols, axis=1)
    idx_ref[0] = idx + b * l


def _run_knn(pos, b, l):
    pos3 = pos.reshape(b, l, 3)
    posT = jnp.transpose(pos3, (0, 2, 1))
    grid = (b, l // _KNN_ROWS)
    return pl.pallas_call(
        functools.partial(_knn_body, l),
        grid=grid,
        in_specs=[
            pl.BlockSpec((1, _KNN_ROWS, 3), lambda bi, t: (bi, t, 0)),
            pl.BlockSpec((1, 3, l), lambda bi, t: (bi, 0, 0)),
        ],
        out_specs=pl.BlockSpec((1, _KNN_ROWS, KK), lambda bi, t: (bi, t, 0)),
        out_shape=jax.ShapeDtypeStruct((b, l, KK), jnp.int32),
    )(pos3, posT)


def _sc_gather(table, idx):
    e = idx.shape[0]
    w = table.shape[1]
    window = 128
    idx2 = idx.reshape(1, e)
    mesh = plsc.VectorSubcoreMesh(core_axis_name="core",
                                  subcore_axis_name="subcore")

    @functools.partial(
        pl.kernel,
        out_type=jax.ShapeDtypeStruct((e, w), table.dtype),
        mesh=mesh,
    )
    def _gather_kernel(x_hbm, i_hbm, o_hbm):
        def body(i_vmem, o_vmem):
            pltpu.sync_copy(x_hbm.at[i_vmem.at[0]], o_vmem)

        pltpu.emit_pipeline(
            body,
            grid=(e // window,),
            in_specs=[pl.BlockSpec((1, window), index_map=lambda i: (0, i))],
            out_specs=[pl.BlockSpec((window, w), index_map=lambda i: (i, 0))],
            core_axis_name=("core", "subcore"),
            dimension_semantics=(pltpu.PARALLEL,),
        )(i_hbm, o_hbm)

    return _gather_kernel(table, idx2)


def _enc_body(g_ref, pos_ref, w1, b1, w2, b2,
              y1_ref, y2_ref, st1_ref, st2_ref):
    t = pl.program_id(0)
    g = g_ref[...]
    rk = g.shape[0]
    r = rk // KK
    pos_i = pos_ref[...]
    pj = g[:, 0:3]
    pi = jnp.broadcast_to(pos_i[:, None, :], (r, KK, 3)).reshape(rk, 3)
    diff = pj - pi
    dist = jnp.sqrt(jnp.sum(diff * diff, axis=1, keepdims=True))
    rel = jnp.concatenate([pi, pj, diff, dist], axis=1)
    y1 = _mm(rel, w1[...]) + b1[...]
    y2 = _mm(rel, w2[...]) + b2[...]
    y1_ref[...] = y1
    y2_ref[...] = y2
    _stats_update(st1_ref, y1, t)
    _stats_update(st2_ref, y2, t)


def _run_enc(g, pos, p_enc1, p_enc2):
    e, wg = g.shape
    n = pos.shape[0]
    rtile = 256
    etile = rtile * KK
    grid = (e // etile,)
    d1 = p_enc1["W"].shape[1]
    d2 = p_enc2["W"].shape[1]
    return pl.pallas_call(
        _enc_body,
        grid=grid,
        in_specs=[
            pl.BlockSpec((etile, wg), lambda t: (t, 0)),
            pl.BlockSpec((rtile, 3), lambda t: (t, 0)),
            _full_spec(p_enc1["W"].shape), _full_spec((1, d1)),
            _full_spec(p_enc2["W"].shape), _full_spec((1, d2)),
        ],
        out_specs=[
            pl.BlockSpec((etile, d1), lambda t: (t, 0)),
            pl.BlockSpec((etile, d2), lambda t: (t, 0)),
            pl.BlockSpec((8, d1), lambda t: (0, 0)),
            pl.BlockSpec((8, d2), lambda t: (0, 0)),
        ],
        out_shape=[
            jax.ShapeDtypeStruct((e, d1), _F32),
            jax.ShapeDtypeStruct((e, d2), _F32),
            jax.ShapeDtypeStruct((8, d1), _F32),
            jax.ShapeDtypeStruct((8, d2), _F32),
        ],
    )(g, pos, p_enc1["W"], p_enc1["b"].reshape(1, -1),
      p_enc2["W"], p_enc2["b"].reshape(1, -1))


def _lfa_body(n_nodes, n_edges, c0, dh, g_ref, stx_ref, gx, bx,
              ye_ref, ste_ref, ge, be, wa, wp, bp,
              out_ref, stp_ref):
    t = pl.program_id(0)
    xj = _bn_apply(g_ref[...][:, c0:c0 + dh], stx_ref[...], n_nodes,
                   gx[...], bx[...], act=True)
    lse = _bn_apply(ye_ref[...], ste_ref[...], n_edges,
                    ge[...], be[...], act=True)
    local = jnp.concatenate([xj, lse], axis=1)
    d = local.shape[1]
    att = _mm(local, wa[...])
    r = att.shape[0] // KK
    att3 = att.reshape(r, KK, d)
    m = jnp.max(att3, axis=1, keepdims=True)
    ex = jnp.exp(att3 - m)
    s = jnp.sum(ex, axis=1, keepdims=True)
    scores = ex / (s + 1e-16)
    out = jnp.sum(scores * local.reshape(r, KK, d), axis=1)
    yp = _mm(out, wp[...]) + bp[...]
    dp = yp.shape[1]
    padw = out_ref.shape[1] - dp
    if padw:
        yp_pad = jnp.concatenate([yp, jnp.zeros((r, padw), _F32)], axis=1)
    else:
        yp_pad = yp
    out_ref[...] = yp_pad
    _stats_update(stp_ref, yp, t)


def _run_lfa(g, stx, p_x, yenc, stenc, p_enc, p_att, p_post,
             n_nodes, c0, dh, out_w):
    e, wg = g.shape
    rtile = 256
    etile = rtile * KK
    grid = (e // etile,)
    de = yenc.shape[1]
    dp = p_post["W"].shape[1]
    body = functools.partial(_lfa_body, float(n_nodes), float(e), c0, dh)
    return pl.pallas_call(
        body,
        grid=grid,
        in_specs=[
            pl.BlockSpec((etile, wg), lambda t: (t, 0)),
            _full_spec(stx.shape),
            _full_spec((1, dh)), _full_spec((1, dh)),
            pl.BlockSpec((etile, de), lambda t: (t, 0)),
            _full_spec((8, de)),
            _full_spec((1, de)), _full_spec((1, de)),
            _full_spec(p_att["W"].shape),
            _full_spec(p_post["W"].shape), _full_spec((1, dp)),
        ],
        out_specs=[
            pl.BlockSpec((rtile, out_w), lambda t: (t, 0)),
            pl.BlockSpec((8, dp), lambda t: (0, 0)),
        ],
        out_shape=[
            jax.ShapeDtypeStruct((n_nodes, out_w), _F32),
            jax.ShapeDtypeStruct((8, dp), _F32),
        ],
    )(g, stx, p_x["gamma"].reshape(1, -1), p_x["beta"].reshape(1, -1),
      yenc, stenc, p_enc["gamma"].reshape(1, -1), p_enc["beta"].reshape(1, -1),
      p_att["W"], p_post["W"], p_post["b"].reshape(1, -1))


def _bnlin_body(n, din, y_ref, st_ref, g, b, w, bb, yo_ref, sto_ref):
    t = pl.program_id(0)
    h = _bn_apply(y_ref[...][:, :din], st_ref[...], n, g[...], b[...],
                  act=True)
    y = _mm(h, w[...]) + bb[...]
    yo_ref[...] = y
    _stats_update(sto_ref, y, t)


def _run_bnlin(yin, stin, p_in, p_lin):
    n, w_in = yin.shape
    din = p_lin["W"].shape[0]
    dout = p_lin["W"].shape[1]
    tile = 256
    grid = (n // tile,)
    body = functools.partial(_bnlin_body, float(n), din)
    return pl.pallas_call(
        body,
        grid=grid,
        in_specs=[
            pl.BlockSpec((tile, w_in), lambda t: (t, 0)),
            _full_spec(stin.shape),
            _full_spec((1, din)), _full_spec((1, din)),
            _full_spec(p_lin["W"].shape), _full_spec((1, dout)),
        ],
        out_specs=[
            pl.BlockSpec((tile, dout), lambda t: (t, 0)),
            pl.BlockSpec((8, dout), lambda t: (0, 0)),
        ],
        out_shape=[
            jax.ShapeDtypeStruct((n, dout), _F32),
            jax.ShapeDtypeStruct((8, dout), _F32),
        ],
    )(yin, stin, p_in["gamma"].reshape(1, -1), p_in["beta"].reshape(1, -1),
      p_lin["W"], p_lin["b"].reshape(1, -1))


def _res_body(n, ym_ref, stm_ref, gm, bm, ysc_ref, stsc_ref, gs, bs, h_ref):
    a = _bn_apply(ym_ref[...], stm_ref[...], n, gm[...], bm[...], act=False)
    c = _bn_apply(ysc_ref[...], stsc_ref[...], n, gs[...], bs[...], act=False)
    h_ref[...] = _lrelu(a + c)


def _run_res(ym, stm, p_m, ysc, stsc, p_s):
    n, d = ym.shape
    tile = 256
    grid = (n // tile,)
    body = functools.partial(_res_body, float(n))
    return pl.pallas_call(
        body,
        grid=grid,
        in_specs=[
            pl.BlockSpec((tile, d), lambda t: (t, 0)),
            _full_spec(stm.shape),
            _full_spec((1, d)), _full_spec((1, d)),
            pl.BlockSpec((tile, d), lambda t: (t, 0)),
            _full_spec(stsc.shape),
            _full_spec((1, d)), _full_spec((1, d)),
        ],
        out_specs=pl.BlockSpec((tile, d), lambda t: (t, 0)),
        out_shape=jax.ShapeDtypeStruct((n, d), _F32),
    )(ym, stm, p_m["gamma"].reshape(1, -1), p_m["beta"].reshape(1, -1),
      ysc, stsc, p_s["gamma"].reshape(1, -1), p_s["beta"].reshape(1, -1))


def _head_body(h_ref, w1, b1, g1, be1, we1, be_1, ge1, bee1, we2, be_2,
               o_ref):
    h = h_ref[...]
    y = _mm(h, w1[...]) + b1[...]
    m = jnp.mean(y, axis=0)
    v = jnp.mean((y - m) ** 2, axis=0)
    y = (y - m) / jnp.sqrt(v + 1e-6) * g1[...] + be1[...]
    y = _lrelu(y)
    g = jnp.max(y.reshape(BB, -1, y.shape[-1]), axis=1)
    o = _mm(g, we1[...]) + be_1[...]
    m2 = jnp.mean(o, axis=0)
    v2 = jnp.mean((o - m2) ** 2, axis=0)
    o = (o - m2) / jnp.sqrt(v2 + 1e-6) * ge1[...] + bee1[...]
    o = _lrelu(o)
    logits = _mm(o, we2[...]) + be_2[...]
    lmax = jnp.max(logits, axis=-1, keepdims=True)
    s = logits - lmax
    lse = jnp.log(jnp.sum(jnp.exp(s), axis=-1, keepdims=True))
    o_ref[...] = s - lse


def _run_head(h, p1, pe1, pe2):
    return pl.pallas_call(
        _head_body,
        out_shape=jax.ShapeDtypeStruct((BB, NUM_CLASSES), _F32),
    )(h, p1["W"], p1["b"].reshape(1, -1), p1["gamma"].reshape(1, -1),
      p1["beta"].reshape(1, -1),
      pe1["W"], pe1["b"].reshape(1, -1), pe1["gamma"].reshape(1, -1),
      pe1["beta"].reshape(1, -1),
      pe2["W"], pe2["b"].reshape(1, -1))


def _block(p, prep_out, pos, b, l):
    n = b * l
    tab, ysc, st1, stsc = prep_out
    dm = st1.shape[1]

    idx = _run_knn(pos, b, l)
    idx_flat = idx.reshape(-1)

    g1 = _sc_gather(tab, idx_flat)

    yenc1, yenc2, ste1, ste2 = _run_enc(g1, pos, p["lfa1"]["enc"],
                                        p["lfa2"]["enc"])

    d2h = p["lfa2"]["enc"]["W"].shape[1]
    pad2 = 128
    ypost1, stp1 = _run_lfa(
        g1, st1, p["mlp1"], yenc1, ste1, p["lfa1"]["enc"],
        p["lfa1"]["att"], p["lfa1"]["post"],
        n_nodes=n, c0=3, dh=dm, out_w=pad2)

    g2 = _sc_gather(ypost1, idx_flat)

    ypost2, stp2 = _run_lfa(
        g2, stp1, p["lfa1"]["post"], yenc2, ste2, p["lfa2"]["enc"],
        p["lfa2"]["att"], p["lfa2"]["post"],
        n_nodes=n, c0=0, dh=d2h, out_w=2 * d2h)

    ym2, stm2 = _run_bnlin(ypost2, stp2, p["lfa2"]["post"], p["mlp2"])

    h = _run_res(ym2, stm2, p["mlp2"], ysc, stsc, p["shortcut"])
    return h


def kernel(x, pos, batch, params):
    p1 = params["block1"]
    p2 = params["block2"]

    prep1 = _run_prep1(x, pos, params["fc0"], p1["mlp1"], p1["shortcut"],
                       tab_w=128)
    h = _block(p1, prep1, pos, BB, LL)

    h_dec = h[::DEC]
    pos1 = pos[::DEC]

    prep2 = _run_prep2(h_dec, pos1, p2["mlp1"], p2["shortcut"], tab_w=128)
    h2 = _block(p2, prep2, pos1, BB, LL // DEC)

    h_head = h2[::DEC]
    return _run_head(h_head, params["mlp1"], params["end1"], params["end2"])


# speedup vs baseline: 8.4183x; 1.0647x over previous
import functools

import jax
import jax.numpy as jnp
import numpy as np
from jax.experimental import pallas as pl
from jax.experimental.pallas import tpu as pltpu
from jax.experimental.pallas import tpu_sc as plsc

NUM_CLASSES = 40
DEC = 4
KK = 16
BB = 8
LL = 2048
NN = BB * LL

_F32 = jnp.float32


def _mm(a, b):
    return jnp.dot(a, b, preferred_element_type=jnp.float32)


def _lrelu(y):
    return jnp.where(y >= 0, y, 0.2 * y)


def _bn_apply(y, st, n, gamma, beta, act):
    d = y.shape[-1]
    s1 = st[0:1, :d]
    s2 = st[1:2, :d]
    m = s1 / n
    v = s2 / n - m * m
    out = (y - m) / jnp.sqrt(v + 1e-6) * gamma + beta
    if act:
        out = _lrelu(out)
    return out


def _stats_update(st_ref, y, step):
    @pl.when(step == 0)
    def _():
        st_ref[...] = jnp.zeros_like(st_ref)

    d = y.shape[-1]
    s1 = jnp.sum(y, axis=0, keepdims=True)
    s2 = jnp.sum(y * y, axis=0, keepdims=True)
    pad = jnp.zeros((6, d), _F32)
    st_ref[...] += jnp.concatenate([s1, s2, pad], axis=0)


def _full_spec(shape, ndim_grid=1):
    zeros = (0,) * len(shape)
    if ndim_grid == 1:
        return pl.BlockSpec(shape, lambda t: zeros)
    return pl.BlockSpec(shape, lambda *g: zeros)


def _prep1_body(x_ref, pos_ref, w0, b0, wm, bm, ws, bs,
                tab_ref, ysc_ref, st1_ref, stsc_ref):
    t = pl.program_id(0)
    h0 = _mm(x_ref[...], w0[...]) + b0[...]
    y1 = _mm(h0, wm[...]) + bm[...]
    ysc = _mm(h0, ws[...]) + bs[...]
    r = y1.shape[0]
    d = y1.shape[1]
    pos = pos_ref[...]
    np2 = jnp.sum(pos * pos, axis=1, keepdims=True)
    padw = tab_ref.shape[1] - 4 - d
    tab_ref[...] = jnp.concatenate(
        [pos, np2, y1, jnp.zeros((r, padw), _F32)], axis=1)
    ysc_ref[...] = ysc
    _stats_update(st1_ref, y1, t)
    _stats_update(stsc_ref, ysc, t)


def _prep2_body(h_ref, pos_ref, wm, bm, ws, bs,
                tab_ref, ysc_ref, st1_ref, stsc_ref):
    t = pl.program_id(0)
    h = h_ref[...]
    y1 = _mm(h, wm[...]) + bm[...]
    ysc = _mm(h, ws[...]) + bs[...]
    r = y1.shape[0]
    d = y1.shape[1]
    pos = pos_ref[...]
    np2 = jnp.sum(pos * pos, axis=1, keepdims=True)
    padw = tab_ref.shape[1] - 4 - d
    tab_ref[...] = jnp.concatenate(
        [pos, np2, y1, jnp.zeros((r, padw), _F32)], axis=1)
    ysc_ref[...] = ysc
    _stats_update(st1_ref, y1, t)
    _stats_update(stsc_ref, ysc, t)


def _run_prep1(x, pos, p_fc0, p_m, p_s, tab_w):
    n = x.shape[0]
    tile = 256
    grid = (n // tile,)
    dm = p_m["W"].shape[1]
    ds = p_s["W"].shape[1]
    return pl.pallas_call(
        _prep1_body,
        grid=grid,
        in_specs=[
            pl.BlockSpec((tile, x.shape[1]), lambda t: (t, 0)),
            pl.BlockSpec((tile, 3), lambda t: (t, 0)),
            _full_spec(p_fc0["W"].shape), _full_spec((1, p_fc0["W"].shape[1])),
            _full_spec(p_m["W"].shape), _full_spec((1, dm)),
            _full_spec(p_s["W"].shape), _full_spec((1, ds)),
        ],
        out_specs=[
            pl.BlockSpec((tile, tab_w), lambda t: (t, 0)),
            pl.BlockSpec((tile, ds), lambda t: (t, 0)),
            pl.BlockSpec((8, dm), lambda t: (0, 0)),
            pl.BlockSpec((8, ds), lambda t: (0, 0)),
        ],
        out_shape=[
            jax.ShapeDtypeStruct((n, tab_w), _F32),
            jax.ShapeDtypeStruct((n, ds), _F32),
            jax.ShapeDtypeStruct((8, dm), _F32),
            jax.ShapeDtypeStruct((8, ds), _F32),
        ],
    )(x, pos, p_fc0["W"], p_fc0["b"].reshape(1, -1),
      p_m["W"], p_m["b"].reshape(1, -1),
      p_s["W"], p_s["b"].reshape(1, -1))


def _run_prep2(h, pos, p_m, p_s, tab_w):
    n = h.shape[0]
    tile = 256
    grid = (n // tile,)
    dm = p_m["W"].shape[1]
    ds = p_s["W"].shape[1]
    return pl.pallas_call(
        _prep2_body,
        grid=grid,
        in_specs=[
            pl.BlockSpec((tile, h.shape[1]), lambda t: (t, 0)),
            pl.BlockSpec((tile, 3), lambda t: (t, 0)),
            _full_spec(p_m["W"].shape), _full_spec((1, dm)),
            _full_spec(p_s["W"].shape), _full_spec((1, ds)),
        ],
        out_specs=[
            pl.BlockSpec((tile, tab_w), lambda t: (t, 0)),
            pl.BlockSpec((tile, ds), lambda t: (t, 0)),
            pl.BlockSpec((8, dm), lambda t: (0, 0)),
            pl.BlockSpec((8, ds), lambda t: (0, 0)),
        ],
        out_shape=[
            jax.ShapeDtypeStruct((n, tab_w), _F32),
            jax.ShapeDtypeStruct((n, ds), _F32),
            jax.ShapeDtypeStruct((8, dm), _F32),
            jax.ShapeDtypeStruct((8, ds), _F32),
        ],
    )(h, pos, p_m["W"], p_m["b"].reshape(1, -1),
      p_s["W"], p_s["b"].reshape(1, -1))


_KNN_ROWS = 32


def _tree(op, xs):
    while len(xs) > 1:
        nxt = [op(xs[i], xs[i + 1]) for i in range(0, len(xs) - 1, 2)]
        if len(xs) % 2:
            nxt.append(xs[-1])
        xs = nxt
    return xs[0]


def _knn_body(l, pos_ref, posT_ref, idx_ref):
    b = pl.program_id(0)
    tile = pos_ref[0]
    d = None
    for c in range(3):
        tc = tile[:, c:c + 1]
        fc = posT_ref[0, c:c + 1, :]
        dc = (tc - fc) ** 2
        d = dc if d is None else d + dc
    nb = l // 128
    ii = jax.lax.broadcasted_iota(jnp.int32, (_KNN_ROWS, 128), 1).astype(_F32)
    dk = []
    for g in range(nb):
        ds = jnp.maximum(d[:, g * 128:(g + 1) * 128], _F32(1e-30))
        bits = jax.lax.bitcast_convert_type(ds, jnp.int32)
        dk.append(jax.lax.bitcast_convert_type(
            (bits & jnp.int32(~15)) | jnp.int32(g), _F32))
    fold = _tree(jnp.minimum, list(dk))
    big = _F32(3.0e38)
    cols = []
    for it in range(KK):
        m = jnp.min(fold, axis=1, keepdims=True)
        jl = jnp.min(jnp.where(fold == m, ii, _F32(128.0)), axis=1,
                     keepdims=True)
        gv = jax.lax.bitcast_convert_type(m, jnp.int32) & jnp.int32(15)
        cols.append(gv * 128 + jl.astype(jnp.int32))
        if it + 1 < KK:
            fold = None
            for g in range(nb):
                dkg = jnp.where(dk[g] == m, big, dk[g])
                dk[g] = dkg
                fold = dkg if fold is None else jnp.minimum(fold, dkg)
    idx = jnp.concatenate(cols, axis=1)
    idx_ref[0] = idx + b * l


def _run_knn(pos, b, l):
    pos3 = pos.reshape(b, l, 3)
    posT = jnp.transpose(pos3, (0, 2, 1))
    grid = (b, l // _KNN_ROWS)
    return pl.pallas_call(
        functools.partial(_knn_body, l),
        grid=grid,
        in_specs=[
            pl.BlockSpec((1, _KNN_ROWS, 3), lambda bi, t: (bi, t, 0)),
            pl.BlockSpec((1, 3, l), lambda bi, t: (bi, 0, 0)),
        ],
        out_specs=pl.BlockSpec((1, _KNN_ROWS, KK), lambda bi, t: (bi, t, 0)),
        out_shape=jax.ShapeDtypeStruct((b, l, KK), jnp.int32),
    )(pos3, posT)


def _sc_gather(table, idx):
    e = idx.shape[0]
    w = table.shape[1]
    window = 128
    idx2 = idx.reshape(1, e)
    mesh = plsc.VectorSubcoreMesh(core_axis_name="core",
                                  subcore_axis_name="subcore")

    @functools.partial(
        pl.kernel,
        out_type=jax.ShapeDtypeStruct((e, w), table.dtype),
        mesh=mesh,
    )
    def _gather_kernel(x_hbm, i_hbm, o_hbm):
        def body(i_vmem, o_vmem):
            pltpu.sync_copy(x_hbm.at[i_vmem.at[0]], o_vmem)

        pltpu.emit_pipeline(
            body,
            grid=(e // window,),
            in_specs=[pl.BlockSpec((1, window), index_map=lambda i: (0, i))],
            out_specs=[pl.BlockSpec((window, w), index_map=lambda i: (i, 0))],
            core_axis_name=("core", "subcore"),
            dimension_semantics=(pltpu.PARALLEL,),
        )(i_hbm, o_hbm)

    return _gather_kernel(table, idx2)


def _fold_groups(x, d):
    w = x.shape[1]
    while w > d:
        w //= 2
        x = x[:, :w] + x[:, w:2 * w]
    return x


def _col_stats(st_ref, y, step, d):
    @pl.when(step == 0)
    def _():
        st_ref[...] = jnp.zeros_like(st_ref)

    r = y.shape[0]
    ones = jnp.ones((8, r), _F32)
    s1 = _fold_groups(_mm(ones, y)[0:1, :], d)
    s2 = _fold_groups(_mm(ones, y * y)[0:1, :], d)
    pad = jnp.zeros((6, d), _F32)
    st_ref[...] += jnp.concatenate([s1, s2, pad], axis=0)


def _bn_coeffs(st, n, gamma, beta):
    d = gamma.shape[-1]
    s1 = st[0:1, :d]
    s2 = st[1:2, :d]
    m = s1 / n
    v = s2 / n - m * m
    scale = gamma / jnp.sqrt(v + 1e-6)
    shift = beta - m * scale
    return scale, shift


def _np_sel_enc():
    s = np.zeros((KK * 128, 112), np.float32)
    for g in range(KK):
        for c in range(3):
            s[128 * g + c, 16 * c + g] = 1.0
            s[128 * g + c, 48 + 3 * g + c] = 1.0
        s[128 * g + 3, 96 + g] = 1.0
    return jnp.asarray(s)


def _np_tile_eye(d):
    s = np.zeros((d, KK * d), np.float32)
    for k in range(KK):
        for c in range(d):
            s[c, k * d + c] = 1.0
    return jnp.asarray(s)


def _enc_derived(p_enc):
    w = p_enc["W"]
    d = w.shape[1]
    bpart = w[3:6] + w[6:9]
    apart = w[0:3] - w[6:9]
    bk = jnp.kron(jnp.eye(KK, dtype=_F32), bpart)
    ta = jnp.tile(apart, (1, KK))
    dm = jnp.kron(jnp.eye(KK, dtype=_F32), w[9:10])
    bw = jnp.tile(p_enc["b"].reshape(1, -1), (1, KK))
    return bk, ta, dm, bw, d


def _enc_body(g_ref, pos_ref, sall, b1k, ta1, d1m, b1w, b2k, ta2, d2m, b2w,
              y1_ref, y2_ref, st1_ref, st2_ref):
    t = pl.program_id(0)
    sel = _mm(g_ref[...], sall[...])
    pos_i = pos_ref[...]
    np2i = jnp.sum(pos_i * pos_i, axis=1, keepdims=True)
    dot = (sel[:, 0:16] * pos_i[:, 0:1] + sel[:, 16:32] * pos_i[:, 1:2]
           + sel[:, 32:48] * pos_i[:, 2:3])
    dist2 = sel[:, 96:112] - 2.0 * dot + np2i
    dist = jnp.sqrt(jnp.maximum(dist2, 0.0))
    pjint = sel[:, 48:96]
    y1 = _mm(pjint, b1k[...]) + _mm(pos_i, ta1[...]) + _mm(dist, d1m[...]) \
        + b1w[...]
    y2 = _mm(pjint, b2k[...]) + _mm(pos_i, ta2[...]) + _mm(dist, d2m[...]) \
        + b2w[...]
    y1_ref[...] = y1
    y2_ref[...] = y2
    d1 = st1_ref.shape[1]
    d2 = st2_ref.shape[1]
    _col_stats(st1_ref, y1, t, d1)
    _col_stats(st2_ref, y2, t, d2)


def _run_enc(gw, pos, p_enc1, p_enc2):
    n = gw.shape[0]
    rtile = 256
    grid = (n // rtile,)
    b1k, ta1, d1m, b1w, d1 = _enc_derived(p_enc1)
    b2k, ta2, d2m, b2w, d2 = _enc_derived(p_enc2)
    sall = _np_sel_enc()
    args = [gw, pos, sall, b1k, ta1, d1m, b1w, b2k, ta2, d2m, b2w]
    in_specs = [
        pl.BlockSpec((rtile, gw.shape[1]), lambda t: (t, 0)),
        pl.BlockSpec((rtile, 3), lambda t: (t, 0)),
    ] + [_full_spec(a.shape) for a in args[2:]]
    return pl.pallas_call(
        _enc_body,
        grid=grid,
        in_specs=in_specs,
        out_specs=[
            pl.BlockSpec((rtile, KK * d1), lambda t: (t, 0)),
            pl.BlockSpec((rtile, KK * d2), lambda t: (t, 0)),
            pl.BlockSpec((8, d1), lambda t: (0, 0)),
            pl.BlockSpec((8, d2), lambda t: (0, 0)),
        ],
        out_shape=[
            jax.ShapeDtypeStruct((n, KK * d1), _F32),
            jax.ShapeDtypeStruct((n, KK * d2), _F32),
            jax.ShapeDtypeStruct((8, d1), _F32),
            jax.ShapeDtypeStruct((8, d2), _F32),
        ],
    )(*args)


def _np_sel_x(c0, dh):
    s = np.zeros((KK * 128, KK * dh), np.float32)
    for g in range(KK):
        for c in range(dh):
            s[128 * g + c0 + c, g * dh + c] = 1.0
    return jnp.asarray(s)


def _np_place(din, off, d):
    s = np.zeros((KK * din, KK * d), np.float32)
    for k in range(KK):
        for c in range(din):
            s[k * din + c, k * d + off + c] = 1.0
    return jnp.asarray(s)


def _fold_pair(x, d, op):
    while x.shape[1] > d:
        w = x.shape[1] // 2
        x = op(x[:, :w], x[:, w:])
    return x


def _lfa_body(n_nodes, n_edges, dh, g_ref, stx_ref, gx, bx,
              ye_ref, ste_ref, ge, be, sxsel, wax, wae, px, pe, t2,
              wp, bp, out_ref, stp_ref):
    t = pl.program_id(0)
    de = ge.shape[-1]
    d = dh + de
    sx, tx = _bn_coeffs(stx_ref[...], n_nodes, gx[...], bx[...])
    se, te = _bn_coeffs(ste_ref[...], n_edges, ge[...], be[...])
    sxw = jnp.tile(sx, (1, KK))
    txw = jnp.tile(tx, (1, KK))
    sew = jnp.tile(se, (1, KK))
    tew = jnp.tile(te, (1, KK))
    xw = _lrelu(_mm(g_ref[...], sxsel[...]) * sxw + txw)
    lw = _lrelu(ye_ref[...] * sew + tew)
    att = _mm(xw, wax[...]) + _mm(lw, wae[...])
    m = _fold_pair(att, d, jnp.maximum)
    e = jnp.exp(att - _mm(m, t2[...]))
    s = _fold_pair(e, d, jnp.add)
    sc = e * _mm(1.0 / (s + 1e-16), t2[...])
    localw = _mm(xw, px[...]) + _mm(lw, pe[...])
    out = _fold_pair(sc * localw, d, jnp.add)
    yp = _mm(out, wp[...]) + bp[...]
    r, dp = yp.shape
    padw = out_ref.shape[1] - dp
    if padw:
        out_ref[...] = jnp.concatenate(
            [yp, jnp.zeros((r, padw), _F32)], axis=1)
    else:
        out_ref[...] = yp
    _col_stats(stp_ref, yp, t, dp)


def _run_lfa(gw, stx, p_x, yenc, stenc, p_enc, p_att, p_post,
             n_nodes, c0, dh, out_w):
    rtile = 256
    grid = (n_nodes // rtile,)
    de = p_enc["W"].shape[1]
    d = dh + de
    dp = p_post["W"].shape[1]
    wa = p_att["W"]
    eye = jnp.eye(KK, dtype=_F32)
    args = [
        gw, stx,
        p_x["gamma"].reshape(1, -1), p_x["beta"].reshape(1, -1),
        yenc, stenc,
        p_enc["gamma"].reshape(1, -1), p_enc["beta"].reshape(1, -1),
        _np_sel_x(c0, dh),
        jnp.kron(eye, wa[:dh, :]), jnp.kron(eye, wa[dh:, :]),
        _np_place(dh, 0, d), _np_place(de, dh, d), _np_tile_eye(d),
        p_post["W"], p_post["b"].reshape(1, -1),
    ]
    body = functools.partial(_lfa_body, float(n_nodes),
                             float(n_nodes * KK), dh)
    in_specs = [
        pl.BlockSpec((rtile, gw.shape[1]), lambda t: (t, 0)),
        _full_spec(stx.shape),
        _full_spec((1, dh)), _full_spec((1, dh)),
        pl.BlockSpec((rtile, KK * de), lambda t: (t, 0)),
        _full_spec((8, de)),
        _full_spec((1, de)), _full_spec((1, de)),
    ] + [_full_spec(a.shape) for a in args[8:]]
    return pl.pallas_call(
        body,
        grid=grid,
        in_specs=in_specs,
        out_specs=[
            pl.BlockSpec((rtile, out_w), lambda t: (t, 0)),
            pl.BlockSpec((8, dp), lambda t: (0, 0)),
        ],
        out_shape=[
            jax.ShapeDtypeStruct((n_nodes, out_w), _F32),
            jax.ShapeDtypeStruct((8, dp), _F32),
        ],
    )(*args)


def _bnlin_body(n, din, y_ref, st_ref, g, b, w, bb, yo_ref, sto_ref):
    t = pl.program_id(0)
    h = _bn_apply(y_ref[...][:, :din], st_ref[...], n, g[...], b[...],
                  act=True)
    y = _mm(h, w[...]) + bb[...]
    yo_ref[...] = y
    _stats_update(sto_ref, y, t)


def _run_bnlin(yin, stin, p_in, p_lin):
    n, w_in = yin.shape
    din = p_lin["W"].shape[0]
    dout = p_lin["W"].shape[1]
    tile = 256
    grid = (n // tile,)
    body = functools.partial(_bnlin_body, float(n), din)
    return pl.pallas_call(
        body,
        grid=grid,
        in_specs=[
            pl.BlockSpec((tile, w_in), lambda t: (t, 0)),
            _full_spec(stin.shape),
            _full_spec((1, din)), _full_spec((1, din)),
            _full_spec(p_lin["W"].shape), _full_spec((1, dout)),
        ],
        out_specs=[
            pl.BlockSpec((tile, dout), lambda t: (t, 0)),
            pl.BlockSpec((8, dout), lambda t: (0, 0)),
        ],
        out_shape=[
            jax.ShapeDtypeStruct((n, dout), _F32),
            jax.ShapeDtypeStruct((8, dout), _F32),
        ],
    )(yin, stin, p_in["gamma"].reshape(1, -1), p_in["beta"].reshape(1, -1),
      p_lin["W"], p_lin["b"].reshape(1, -1))


def _res_body(n, ym_ref, stm_ref, gm, bm, ysc_ref, stsc_ref, gs, bs, h_ref):
    a = _bn_apply(ym_ref[...], stm_ref[...], n, gm[...], bm[...], act=False)
    c = _bn_apply(ysc_ref[...], stsc_ref[...], n, gs[...], bs[...], act=False)
    h_ref[...] = _lrelu(a + c)


def _run_res(ym, stm, p_m, ysc, stsc, p_s):
    n, d = ym.shape
    tile = 256
    grid = (n // tile,)
    body = functools.partial(_res_body, float(n))
    return pl.pallas_call(
        body,
        grid=grid,
        in_specs=[
            pl.BlockSpec((tile, d), lambda t: (t, 0)),
            _full_spec(stm.shape),
            _full_spec((1, d)), _full_spec((1, d)),
            pl.BlockSpec((tile, d), lambda t: (t, 0)),
            _full_spec(stsc.shape),
            _full_spec((1, d)), _full_spec((1, d)),
        ],
        out_specs=pl.BlockSpec((tile, d), lambda t: (t, 0)),
        out_shape=jax.ShapeDtypeStruct((n, d), _F32),
    )(ym, stm, p_m["gamma"].reshape(1, -1), p_m["beta"].reshape(1, -1),
      ysc, stsc, p_s["gamma"].reshape(1, -1), p_s["beta"].reshape(1, -1))


def _head_body(h_ref, w1, b1, g1, be1, we1, be_1, ge1, bee1, we2, be_2,
               o_ref):
    h = h_ref[...]
    y = _mm(h, w1[...]) + b1[...]
    m = jnp.mean(y, axis=0)
    v = jnp.mean((y - m) ** 2, axis=0)
    y = (y - m) / jnp.sqrt(v + 1e-6) * g1[...] + be1[...]
    y = _lrelu(y)
    g = jnp.max(y.reshape(BB, -1, y.shape[-1]), axis=1)
    o = _mm(g, we1[...]) + be_1[...]
    m2 = jnp.mean(o, axis=0)
    v2 = jnp.mean((o - m2) ** 2, axis=0)
    o = (o - m2) / jnp.sqrt(v2 + 1e-6) * ge1[...] + bee1[...]
    o = _lrelu(o)
    logits = _mm(o, we2[...]) + be_2[...]
    lmax = jnp.max(logits, axis=-1, keepdims=True)
    s = logits - lmax
    lse = jnp.log(jnp.sum(jnp.exp(s), axis=-1, keepdims=True))
    o_ref[...] = s - lse


def _run_head(h, p1, pe1, pe2):
    return pl.pallas_call(
        _head_body,
        out_shape=jax.ShapeDtypeStruct((BB, NUM_CLASSES), _F32),
    )(h, p1["W"], p1["b"].reshape(1, -1), p1["gamma"].reshape(1, -1),
      p1["beta"].reshape(1, -1),
      pe1["W"], pe1["b"].reshape(1, -1), pe1["gamma"].reshape(1, -1),
      pe1["beta"].reshape(1, -1),
      pe2["W"], pe2["b"].reshape(1, -1))


def _block(p, prep_out, pos, b, l):
    n = b * l
    tab, ysc, st1, stsc = prep_out
    dm = st1.shape[1]

    idx = _run_knn(pos, b, l)
    idx_flat = idx.reshape(-1)

    g1 = _sc_gather(tab, idx_flat)
    g1w = g1.reshape(n, KK * 128)

    yenc1, yenc2, ste1, ste2 = _run_enc(g1w, pos, p["lfa1"]["enc"],
                                        p["lfa2"]["enc"])

    d2h = p["lfa2"]["enc"]["W"].shape[1]
    ypost1, stp1 = _run_lfa(
        g1w, st1, p["mlp1"], yenc1, ste1, p["lfa1"]["enc"],
        p["lfa1"]["att"], p["lfa1"]["post"],
        n_nodes=n, c0=4, dh=dm, out_w=128)

    g2 = _sc_gather(ypost1, idx_flat)
    g2w = g2.reshape(n, KK * 128)

    ypost2, stp2 = _run_lfa(
        g2w, stp1, p["lfa1"]["post"], yenc2, ste2, p["lfa2"]["enc"],
        p["lfa2"]["att"], p["lfa2"]["post"],
        n_nodes=n, c0=0, dh=d2h, out_w=2 * d2h)

    ym2, stm2 = _run_bnlin(ypost2, stp2, p["lfa2"]["post"], p["mlp2"])

    h = _run_res(ym2, stm2, p["mlp2"], ysc, stsc, p["shortcut"])
    return h


def kernel(x, pos, batch, params):
    p1 = params["block1"]
    p2 = params["block2"]

    prep1 = _run_prep1(x, pos, params["fc0"], p1["mlp1"], p1["shortcut"],
                       tab_w=128)
    h = _block(p1, prep1, pos, BB, LL)

    h_dec = h[::DEC]
    pos1 = pos[::DEC]

    prep2 = _run_prep2(h_dec, pos1, p2["mlp1"], p2["shortcut"], tab_w=128)
    h2 = _block(p2, prep2, pos1, BB, LL // DEC)

    h_head = h2[::DEC]
    return _run_head(h_head, params["mlp1"], params["end1"], params["end2"])


# speedup vs baseline: 11.6205x; 1.3804x over previous
import functools

import jax
import jax.numpy as jnp
import numpy as np
from jax.experimental import pallas as pl
from jax.experimental.pallas import tpu as pltpu
from jax.experimental.pallas import tpu_sc as plsc

NUM_CLASSES = 40
DEC = 4
KK = 16
BB = 8
LL = 2048
NN = BB * LL

_F32 = jnp.float32


def _mm(a, b):
    return jnp.dot(a, b, preferred_element_type=jnp.float32)


def _lrelu(y):
    return jnp.where(y >= 0, y, 0.2 * y)


def _bn_apply(y, st, n, gamma, beta, act):
    d = y.shape[-1]
    s1 = st[0:1, :d]
    s2 = st[1:2, :d]
    m = s1 / n
    v = s2 / n - m * m
    out = (y - m) / jnp.sqrt(v + 1e-6) * gamma + beta
    if act:
        out = _lrelu(out)
    return out


def _stats_update(st_ref, y, step):
    @pl.when(step == 0)
    def _():
        st_ref[...] = jnp.zeros_like(st_ref)

    d = y.shape[-1]
    s1 = jnp.sum(y, axis=0, keepdims=True)
    s2 = jnp.sum(y * y, axis=0, keepdims=True)
    pad = jnp.zeros((6, d), _F32)
    st_ref[...] += jnp.concatenate([s1, s2, pad], axis=0)


def _full_spec(shape, ndim_grid=1):
    zeros = (0,) * len(shape)
    if ndim_grid == 1:
        return pl.BlockSpec(shape, lambda t: zeros)
    return pl.BlockSpec(shape, lambda *g: zeros)


def _prep1_body(x_ref, pos_ref, w0, b0, wm, bm, ws, bs,
                tab_ref, ysc_ref, st1_ref, stsc_ref):
    t = pl.program_id(0)
    h0 = _mm(x_ref[...], w0[...]) + b0[...]
    y1 = _mm(h0, wm[...]) + bm[...]
    ysc = _mm(h0, ws[...]) + bs[...]
    r = y1.shape[0]
    d = y1.shape[1]
    pos = pos_ref[...]
    np2 = jnp.sum(pos * pos, axis=1, keepdims=True)
    padw = tab_ref.shape[1] - 4 - d
    tab_ref[...] = jnp.concatenate(
        [pos, np2, y1, jnp.zeros((r, padw), _F32)], axis=1)
    ysc_ref[...] = ysc
    _stats_update(st1_ref, y1, t)
    _stats_update(stsc_ref, ysc, t)


def _prep2_body(h_ref, pos_ref, wm, bm, ws, bs,
                tab_ref, ysc_ref, st1_ref, stsc_ref):
    t = pl.program_id(0)
    h = h_ref[...]
    y1 = _mm(h, wm[...]) + bm[...]
    ysc = _mm(h, ws[...]) + bs[...]
    r = y1.shape[0]
    d = y1.shape[1]
    pos = pos_ref[...]
    np2 = jnp.sum(pos * pos, axis=1, keepdims=True)
    padw = tab_ref.shape[1] - 4 - d
    tab_ref[...] = jnp.concatenate(
        [pos, np2, y1, jnp.zeros((r, padw), _F32)], axis=1)
    ysc_ref[...] = ysc
    _stats_update(st1_ref, y1, t)
    _stats_update(stsc_ref, ysc, t)


def _run_prep1(x, pos, p_fc0, p_m, p_s, tab_w):
    n = x.shape[0]
    tile = 256
    grid = (n // tile,)
    dm = p_m["W"].shape[1]
    ds = p_s["W"].shape[1]
    return pl.pallas_call(
        _prep1_body,
        grid=grid,
        in_specs=[
            pl.BlockSpec((tile, x.shape[1]), lambda t: (t, 0)),
            pl.BlockSpec((tile, 3), lambda t: (t, 0)),
            _full_spec(p_fc0["W"].shape), _full_spec((1, p_fc0["W"].shape[1])),
            _full_spec(p_m["W"].shape), _full_spec((1, dm)),
            _full_spec(p_s["W"].shape), _full_spec((1, ds)),
        ],
        out_specs=[
            pl.BlockSpec((tile, tab_w), lambda t: (t, 0)),
            pl.BlockSpec((tile, ds), lambda t: (t, 0)),
            pl.BlockSpec((8, dm), lambda t: (0, 0)),
            pl.BlockSpec((8, ds), lambda t: (0, 0)),
        ],
        out_shape=[
            jax.ShapeDtypeStruct((n, tab_w), _F32),
            jax.ShapeDtypeStruct((n, ds), _F32),
            jax.ShapeDtypeStruct((8, dm), _F32),
            jax.ShapeDtypeStruct((8, ds), _F32),
        ],
    )(x, pos, p_fc0["W"], p_fc0["b"].reshape(1, -1),
      p_m["W"], p_m["b"].reshape(1, -1),
      p_s["W"], p_s["b"].reshape(1, -1))


def _run_prep2(h, pos, p_m, p_s, tab_w):
    n = h.shape[0]
    tile = 256
    grid = (n // tile,)
    dm = p_m["W"].shape[1]
    ds = p_s["W"].shape[1]
    return pl.pallas_call(
        _prep2_body,
        grid=grid,
        in_specs=[
            pl.BlockSpec((tile, h.shape[1]), lambda t: (t, 0)),
            pl.BlockSpec((tile, 3), lambda t: (t, 0)),
            _full_spec(p_m["W"].shape), _full_spec((1, dm)),
            _full_spec(p_s["W"].shape), _full_spec((1, ds)),
        ],
        out_specs=[
            pl.BlockSpec((tile, tab_w), lambda t: (t, 0)),
            pl.BlockSpec((tile, ds), lambda t: (t, 0)),
            pl.BlockSpec((8, dm), lambda t: (0, 0)),
            pl.BlockSpec((8, ds), lambda t: (0, 0)),
        ],
        out_shape=[
            jax.ShapeDtypeStruct((n, tab_w), _F32),
            jax.ShapeDtypeStruct((n, ds), _F32),
            jax.ShapeDtypeStruct((8, dm), _F32),
            jax.ShapeDtypeStruct((8, ds), _F32),
        ],
    )(h, pos, p_m["W"], p_m["b"].reshape(1, -1),
      p_s["W"], p_s["b"].reshape(1, -1))


_KNN_ROWS = 128


def _tree(op, xs):
    while len(xs) > 1:
        nxt = [op(xs[i], xs[i + 1]) for i in range(0, len(xs) - 1, 2)]
        if len(xs) % 2:
            nxt.append(xs[-1])
        xs = nxt
    return xs[0]


def _knn_body(l, pos_ref, posT_ref, idx_ref):
    b = pl.program_id(0)
    tile = pos_ref[0]
    d = None
    for c in range(3):
        tc = tile[:, c:c + 1]
        fc = posT_ref[0, c:c + 1, :]
        dc = (tc - fc) ** 2
        d = dc if d is None else d + dc
    nb = l // 128
    ii = jax.lax.broadcasted_iota(jnp.int32, (_KNN_ROWS, 128), 1).astype(_F32)
    dk = []
    for g in range(nb):
        ds = jnp.maximum(d[:, g * 128:(g + 1) * 128], _F32(1e-30))
        bits = jax.lax.bitcast_convert_type(ds, jnp.int32)
        dk.append(jax.lax.bitcast_convert_type(
            (bits & jnp.int32(~15)) | jnp.int32(g), _F32))
    fold = _tree(jnp.minimum, list(dk))
    big = _F32(3.0e38)
    cols = []
    for it in range(KK):
        m = jnp.min(fold, axis=1, keepdims=True)
        jl = jnp.min(jnp.where(fold == m, ii, _F32(128.0)), axis=1,
                     keepdims=True)
        gv = jax.lax.bitcast_convert_type(m, jnp.int32) & jnp.int32(15)
        cols.append(gv * 128 + jl.astype(jnp.int32))
        if it + 1 < KK:
            fold = None
            for g in range(nb):
                dkg = jnp.where(dk[g] == m, big, dk[g])
                dk[g] = dkg
                fold = dkg if fold is None else jnp.minimum(fold, dkg)
    idx = jnp.concatenate(cols, axis=1)
    idx_ref[0] = idx + b * l


def _run_knn(pos, b, l):
    pos3 = pos.reshape(b, l, 3)
    posT = jnp.transpose(pos3, (0, 2, 1))
    grid = (b, l // _KNN_ROWS)
    return pl.pallas_call(
        functools.partial(_knn_body, l),
        grid=grid,
        in_specs=[
            pl.BlockSpec((1, _KNN_ROWS, 3), lambda bi, t: (bi, t, 0)),
            pl.BlockSpec((1, 3, l), lambda bi, t: (bi, 0, 0)),
        ],
        out_specs=pl.BlockSpec((1, _KNN_ROWS, KK), lambda bi, t: (bi, t, 0)),
        out_shape=jax.ShapeDtypeStruct((b, l, KK), jnp.int32),
    )(pos3, posT)


def _f32_to_i8(t):
    n, w = t.shape
    return jax.lax.bitcast_convert_type(t, jnp.int8).reshape(n, 4 * w)


def _i8_to_f32(t):
    n, w = t.shape
    return jax.lax.bitcast_convert_type(
        t.reshape(n, w // 4, 4), jnp.float32)


def _sc_gather(table, idx):
    e = idx.shape[0]
    w = table.shape[1]
    window = 128
    idx2 = idx.reshape(1, e)
    mesh = plsc.VectorSubcoreMesh(core_axis_name="core",
                                  subcore_axis_name="subcore")

    @functools.partial(
        pl.kernel,
        out_type=jax.ShapeDtypeStruct((e, w), table.dtype),
        mesh=mesh,
    )
    def _gather_kernel(x_hbm, i_hbm, o_hbm):
        def body(i_vmem, o_vmem):
            pltpu.sync_copy(x_hbm.at[i_vmem.at[0]], o_vmem)

        pltpu.emit_pipeline(
            body,
            grid=(e // window,),
            in_specs=[pl.BlockSpec((1, window), index_map=lambda i: (0, i))],
            out_specs=[pl.BlockSpec((window, w), index_map=lambda i: (i, 0))],
            core_axis_name=("core", "subcore"),
            dimension_semantics=(pltpu.PARALLEL,),
        )(i_hbm, o_hbm)

    return _gather_kernel(table, idx2)


def _fold_groups(x, d):
    w = x.shape[1]
    while w > d:
        w //= 2
        x = x[:, :w] + x[:, w:2 * w]
    return x


def _col_stats(st_ref, y, step, d):
    @pl.when(step == 0)
    def _():
        st_ref[...] = jnp.zeros_like(st_ref)

    r = y.shape[0]
    ones = jnp.ones((8, r), _F32)
    s1 = _fold_groups(_mm(ones, y)[0:1, :], d)
    s2 = _fold_groups(_mm(ones, y * y)[0:1, :], d)
    pad = jnp.zeros((6, d), _F32)
    st_ref[...] += jnp.concatenate([s1, s2, pad], axis=0)


def _bn_coeffs(st, n, gamma, beta):
    d = gamma.shape[-1]
    s1 = st[0:1, :d]
    s2 = st[1:2, :d]
    m = s1 / n
    v = s2 / n - m * m
    scale = gamma / jnp.sqrt(v + 1e-6)
    shift = beta - m * scale
    return scale, shift


_TABW = 128


def _np_sel_enc():
    s = np.zeros((KK * _TABW, 112), np.float32)
    for g in range(KK):
        for c in range(3):
            s[_TABW * g + c, 16 * c + g] = 1.0
            s[_TABW * g + c, 48 + 3 * g + c] = 1.0
        s[_TABW * g + 3, 96 + g] = 1.0
    return jnp.asarray(s)


def _np_tile_eye(d):
    s = np.zeros((d, KK * d), np.float32)
    for k in range(KK):
        for c in range(d):
            s[c, k * d + c] = 1.0
    return jnp.asarray(s)


def _enc_derived(p_enc):
    w = p_enc["W"]
    d = w.shape[1]
    bpart = w[3:6] + w[6:9]
    apart = w[0:3] - w[6:9]
    bk = jnp.kron(jnp.eye(KK, dtype=_F32), bpart)
    ta = jnp.tile(apart, (1, KK))
    dm = jnp.kron(jnp.eye(KK, dtype=_F32), w[9:10])
    bw = jnp.tile(p_enc["b"].reshape(1, -1), (1, KK))
    return bk, ta, dm, bw, d


def _enc_body(g_ref, pos_ref, sall, b1k, ta1, d1m, b1w, b2k, ta2, d2m, b2w,
              y1_ref, y2_ref, st1_ref, st2_ref):
    t = pl.program_id(0)
    sel = _mm(g_ref[...], sall[...])
    pos_i = pos_ref[...]
    np2i = jnp.sum(pos_i * pos_i, axis=1, keepdims=True)
    dot = (sel[:, 0:16] * pos_i[:, 0:1] + sel[:, 16:32] * pos_i[:, 1:2]
           + sel[:, 32:48] * pos_i[:, 2:3])
    dist2 = sel[:, 96:112] - 2.0 * dot + np2i
    dist = jnp.sqrt(jnp.maximum(dist2, 0.0))
    pjint = sel[:, 48:96]
    y1 = _mm(pjint, b1k[...]) + _mm(pos_i, ta1[...]) + _mm(dist, d1m[...]) \
        + b1w[...]
    y2 = _mm(pjint, b2k[...]) + _mm(pos_i, ta2[...]) + _mm(dist, d2m[...]) \
        + b2w[...]
    y1_ref[...] = y1
    y2_ref[...] = y2
    d1 = st1_ref.shape[1]
    d2 = st2_ref.shape[1]
    _col_stats(st1_ref, y1, t, d1)
    _col_stats(st2_ref, y2, t, d2)


def _run_enc(gw, pos, p_enc1, p_enc2):
    n = gw.shape[0]
    rtile = 256
    grid = (n // rtile,)
    b1k, ta1, d1m, b1w, d1 = _enc_derived(p_enc1)
    b2k, ta2, d2m, b2w, d2 = _enc_derived(p_enc2)
    sall = _np_sel_enc()
    args = [gw, pos, sall, b1k, ta1, d1m, b1w, b2k, ta2, d2m, b2w]
    in_specs = [
        pl.BlockSpec((rtile, gw.shape[1]), lambda t: (t, 0)),
        pl.BlockSpec((rtile, 3), lambda t: (t, 0)),
    ] + [_full_spec(a.shape) for a in args[2:]]
    return pl.pallas_call(
        _enc_body,
        grid=grid,
        in_specs=in_specs,
        out_specs=[
            pl.BlockSpec((rtile, KK * d1), lambda t: (t, 0)),
            pl.BlockSpec((rtile, KK * d2), lambda t: (t, 0)),
            pl.BlockSpec((8, d1), lambda t: (0, 0)),
            pl.BlockSpec((8, d2), lambda t: (0, 0)),
        ],
        out_shape=[
            jax.ShapeDtypeStruct((n, KK * d1), _F32),
            jax.ShapeDtypeStruct((n, KK * d2), _F32),
            jax.ShapeDtypeStruct((8, d1), _F32),
            jax.ShapeDtypeStruct((8, d2), _F32),
        ],
    )(*args)


def _np_sel_x(c0, dh):
    s = np.zeros((KK * _TABW, KK * dh), np.float32)
    for g in range(KK):
        for c in range(dh):
            s[_TABW * g + c0 + c, g * dh + c] = 1.0
    return jnp.asarray(s)


def _np_place(din, off, d):
    s = np.zeros((KK * din, KK * d), np.float32)
    for k in range(KK):
        for c in range(din):
            s[k * din + c, k * d + off + c] = 1.0
    return jnp.asarray(s)


def _fold_pair(x, d, op):
    while x.shape[1] > d:
        w = x.shape[1] // 2
        x = op(x[:, :w], x[:, w:])
    return x


def _lfa_body(n_nodes, n_edges, dh, g_ref, stx_ref, gx, bx,
              ye_ref, ste_ref, ge, be, sxsel, wax, wae, px, pe, t2,
              wp, bp, out_ref, stp_ref):
    t = pl.program_id(0)
    de = ge.shape[-1]
    d = dh + de
    sx, tx = _bn_coeffs(stx_ref[...], n_nodes, gx[...], bx[...])
    se, te = _bn_coeffs(ste_ref[...], n_edges, ge[...], be[...])
    sxw = jnp.tile(sx, (1, KK))
    txw = jnp.tile(tx, (1, KK))
    sew = jnp.tile(se, (1, KK))
    tew = jnp.tile(te, (1, KK))
    xw = _lrelu(_mm(g_ref[...], sxsel[...]) * sxw + txw)
    lw = _lrelu(ye_ref[...] * sew + tew)
    att = _mm(xw, wax[...]) + _mm(lw, wae[...])
    m = _fold_pair(att, d, jnp.maximum)
    e = jnp.exp(att - _mm(m, t2[...]))
    s = _fold_pair(e, d, jnp.add)
    sc = e * _mm(1.0 / (s + 1e-16), t2[...])
    localw = _mm(xw, px[...]) + _mm(lw, pe[...])
    out = _fold_pair(sc * localw, d, jnp.add)
    yp = _mm(out, wp[...]) + bp[...]
    r, dp = yp.shape
    padw = out_ref.shape[1] - dp
    if padw:
        out_ref[...] = jnp.concatenate(
            [yp, jnp.zeros((r, padw), _F32)], axis=1)
    else:
        out_ref[...] = yp
    _col_stats(stp_ref, yp, t, dp)


def _run_lfa(gw, stx, p_x, yenc, stenc, p_enc, p_att, p_post,
             n_nodes, c0, dh, out_w):
    rtile = 256
    grid = (n_nodes // rtile,)
    de = p_enc["W"].shape[1]
    d = dh + de
    dp = p_post["W"].shape[1]
    wa = p_att["W"]
    eye = jnp.eye(KK, dtype=_F32)
    args = [
        gw, stx,
        p_x["gamma"].reshape(1, -1), p_x["beta"].reshape(1, -1),
        yenc, stenc,
        p_enc["gamma"].reshape(1, -1), p_enc["beta"].reshape(1, -1),
        _np_sel_x(c0, dh),
        jnp.kron(eye, wa[:dh, :]), jnp.kron(eye, wa[dh:, :]),
        _np_place(dh, 0, d), _np_place(de, dh, d), _np_tile_eye(d),
        p_post["W"], p_post["b"].reshape(1, -1),
    ]
    body = functools.partial(_lfa_body, float(n_nodes),
                             float(n_nodes * KK), dh)
    in_specs = [
        pl.BlockSpec((rtile, gw.shape[1]), lambda t: (t, 0)),
        _full_spec(stx.shape),
        _full_spec((1, dh)), _full_spec((1, dh)),
        pl.BlockSpec((rtile, KK * de), lambda t: (t, 0)),
        _full_spec((8, de)),
        _full_spec((1, de)), _full_spec((1, de)),
    ] + [_full_spec(a.shape) for a in args[8:]]
    return pl.pallas_call(
        body,
        grid=grid,
        in_specs=in_specs,
        out_specs=[
            pl.BlockSpec((rtile, out_w), lambda t: (t, 0)),
            pl.BlockSpec((8, dp), lambda t: (0, 0)),
        ],
        out_shape=[
            jax.ShapeDtypeStruct((n_nodes, out_w), _F32),
            jax.ShapeDtypeStruct((8, dp), _F32),
        ],
    )(*args)


def _bnlin_body(n, din, y_ref, st_ref, g, b, w, bb, yo_ref, sto_ref):
    t = pl.program_id(0)
    h = _bn_apply(y_ref[...][:, :din], st_ref[...], n, g[...], b[...],
                  act=True)
    y = _mm(h, w[...]) + bb[...]
    yo_ref[...] = y
    _stats_update(sto_ref, y, t)


def _run_bnlin(yin, stin, p_in, p_lin):
    n, w_in = yin.shape
    din = p_lin["W"].shape[0]
    dout = p_lin["W"].shape[1]
    tile = 256
    grid = (n // tile,)
    body = functools.partial(_bnlin_body, float(n), din)
    return pl.pallas_call(
        body,
        grid=grid,
        in_specs=[
            pl.BlockSpec((tile, w_in), lambda t: (t, 0)),
            _full_spec(stin.shape),
            _full_spec((1, din)), _full_spec((1, din)),
            _full_spec(p_lin["W"].shape), _full_spec((1, dout)),
        ],
        out_specs=[
            pl.BlockSpec((tile, dout), lambda t: (t, 0)),
            pl.BlockSpec((8, dout), lambda t: (0, 0)),
        ],
        out_shape=[
            jax.ShapeDtypeStruct((n, dout), _F32),
            jax.ShapeDtypeStruct((8, dout), _F32),
        ],
    )(yin, stin, p_in["gamma"].reshape(1, -1), p_in["beta"].reshape(1, -1),
      p_lin["W"], p_lin["b"].reshape(1, -1))


def _res_body(n, ym_ref, stm_ref, gm, bm, ysc_ref, stsc_ref, gs, bs, h_ref):
    a = _bn_apply(ym_ref[...], stm_ref[...], n, gm[...], bm[...], act=False)
    c = _bn_apply(ysc_ref[...], stsc_ref[...], n, gs[...], bs[...], act=False)
    h_ref[...] = _lrelu(a + c)


def _run_res(ym, stm, p_m, ysc, stsc, p_s):
    n, d = ym.shape
    tile = 256
    grid = (n // tile,)
    body = functools.partial(_res_body, float(n))
    return pl.pallas_call(
        body,
        grid=grid,
        in_specs=[
            pl.BlockSpec((tile, d), lambda t: (t, 0)),
            _full_spec(stm.shape),
            _full_spec((1, d)), _full_spec((1, d)),
            pl.BlockSpec((tile, d), lambda t: (t, 0)),
            _full_spec(stsc.shape),
            _full_spec((1, d)), _full_spec((1, d)),
        ],
        out_specs=pl.BlockSpec((tile, d), lambda t: (t, 0)),
        out_shape=jax.ShapeDtypeStruct((n, d), _F32),
    )(ym, stm, p_m["gamma"].reshape(1, -1), p_m["beta"].reshape(1, -1),
      ysc, stsc, p_s["gamma"].reshape(1, -1), p_s["beta"].reshape(1, -1))


def _head_body(h_ref, w1, b1, g1, be1, we1, be_1, ge1, bee1, we2, be_2,
               o_ref):
    h = h_ref[...]
    y = _mm(h, w1[...]) + b1[...]
    m = jnp.mean(y, axis=0)
    v = jnp.mean((y - m) ** 2, axis=0)
    y = (y - m) / jnp.sqrt(v + 1e-6) * g1[...] + be1[...]
    y = _lrelu(y)
    g = jnp.max(y.reshape(BB, -1, y.shape[-1]), axis=1)
    o = _mm(g, we1[...]) + be_1[...]
    m2 = jnp.mean(o, axis=0)
    v2 = jnp.mean((o - m2) ** 2, axis=0)
    o = (o - m2) / jnp.sqrt(v2 + 1e-6) * ge1[...] + bee1[...]
    o = _lrelu(o)
    logits = _mm(o, we2[...]) + be_2[...]
    lmax = jnp.max(logits, axis=-1, keepdims=True)
    s = logits - lmax
    lse = jnp.log(jnp.sum(jnp.exp(s), axis=-1, keepdims=True))
    o_ref[...] = s - lse


def _run_head(h, p1, pe1, pe2):
    return pl.pallas_call(
        _head_body,
        out_shape=jax.ShapeDtypeStruct((BB, NUM_CLASSES), _F32),
    )(h, p1["W"], p1["b"].reshape(1, -1), p1["gamma"].reshape(1, -1),
      p1["beta"].reshape(1, -1),
      pe1["W"], pe1["b"].reshape(1, -1), pe1["gamma"].reshape(1, -1),
      pe1["beta"].reshape(1, -1),
      pe2["W"], pe2["b"].reshape(1, -1))


def _block(p, prep_out, pos, b, l):
    n = b * l
    tab, ysc, st1, stsc = prep_out
    dm = st1.shape[1]

    idx = _run_knn(pos, b, l)
    idx_flat = idx.reshape(-1)

    g1 = _sc_gather(tab, idx_flat)
    g1w = g1.reshape(n, KK * _TABW)

    yenc1, yenc2, ste1, ste2 = _run_enc(g1w, pos, p["lfa1"]["enc"],
                                        p["lfa2"]["enc"])

    d2h = p["lfa2"]["enc"]["W"].shape[1]
    ypost1, stp1 = _run_lfa(
        g1w, st1, p["mlp1"], yenc1, ste1, p["lfa1"]["enc"],
        p["lfa1"]["att"], p["lfa1"]["post"],
        n_nodes=n, c0=4, dh=dm, out_w=_TABW)

    g2 = _sc_gather(ypost1, idx_flat)
    g2w = g2.reshape(n, KK * _TABW)

    ypost2, stp2 = _run_lfa(
        g2w, stp1, p["lfa1"]["post"], yenc2, ste2, p["lfa2"]["enc"],
        p["lfa2"]["att"], p["lfa2"]["post"],
        n_nodes=n, c0=0, dh=d2h, out_w=2 * d2h)

    ym2, stm2 = _run_bnlin(ypost2, stp2, p["lfa2"]["post"], p["mlp2"])

    h = _run_res(ym2, stm2, p["mlp2"], ysc, stsc, p["shortcut"])
    return h


def kernel(x, pos, batch, params):
    p1 = params["block1"]
    p2 = params["block2"]

    prep1 = _run_prep1(x, pos, params["fc0"], p1["mlp1"], p1["shortcut"],
                       tab_w=_TABW)
    h = _block(p1, prep1, pos, BB, LL)

    h_dec = h[::DEC]
    pos1 = pos[::DEC]

    prep2 = _run_prep2(h_dec, pos1, p2["mlp1"], p2["shortcut"], tab_w=_TABW)
    h2 = _block(p2, prep2, pos1, BB, LL // DEC)

    h_head = h2[::DEC]
    return _run_head(h_head, params["mlp1"], params["end1"], params["end2"])


# speedup vs baseline: 11.6922x; 1.0062x over previous
import functools

import jax
import jax.numpy as jnp
import numpy as np
from jax.experimental import pallas as pl
from jax.experimental.pallas import tpu as pltpu
from jax.experimental.pallas import tpu_sc as plsc

NUM_CLASSES = 40
DEC = 4
KK = 16
BB = 8
LL = 2048
NN = BB * LL

_F32 = jnp.float32


def _mm(a, b):
    return jnp.dot(a, b, preferred_element_type=jnp.float32)


def _lrelu(y):
    return jnp.where(y >= 0, y, 0.2 * y)


def _bn_apply(y, st, n, gamma, beta, act):
    d = y.shape[-1]
    s1 = st[0:1, :d]
    s2 = st[1:2, :d]
    m = s1 / n
    v = s2 / n - m * m
    out = (y - m) / jnp.sqrt(v + 1e-6) * gamma + beta
    if act:
        out = _lrelu(out)
    return out


def _stats_update(st_ref, y, step):
    @pl.when(step == 0)
    def _():
        st_ref[...] = jnp.zeros_like(st_ref)

    d = y.shape[-1]
    s1 = jnp.sum(y, axis=0, keepdims=True)
    s2 = jnp.sum(y * y, axis=0, keepdims=True)
    pad = jnp.zeros((6, d), _F32)
    st_ref[...] += jnp.concatenate([s1, s2, pad], axis=0)


def _full_spec(shape, ndim_grid=1):
    zeros = (0,) * len(shape)
    if ndim_grid == 1:
        return pl.BlockSpec(shape, lambda t: zeros)
    return pl.BlockSpec(shape, lambda *g: zeros)


def _prep1_body(x_ref, pos_ref, w0, b0, wm, bm, ws, bs,
                tab_ref, ysc_ref, st1_ref, stsc_ref):
    t = pl.program_id(0)
    h0 = _mm(x_ref[...], w0[...]) + b0[...]
    y1 = _mm(h0, wm[...]) + bm[...]
    ysc = _mm(h0, ws[...]) + bs[...]
    r = y1.shape[0]
    d = y1.shape[1]
    pos = pos_ref[...]
    np2 = jnp.sum(pos * pos, axis=1, keepdims=True)
    padw = tab_ref.shape[1] - 4 - d
    tab_ref[...] = jnp.concatenate(
        [pos, np2, y1, jnp.zeros((r, padw), _F32)], axis=1)
    ysc_ref[...] = ysc
    _stats_update(st1_ref, y1, t)
    _stats_update(stsc_ref, ysc, t)


def _prep2_body(h_ref, pos_ref, wm, bm, ws, bs,
                tab_ref, ysc_ref, st1_ref, stsc_ref):
    t = pl.program_id(0)
    h = h_ref[...]
    y1 = _mm(h, wm[...]) + bm[...]
    ysc = _mm(h, ws[...]) + bs[...]
    r = y1.shape[0]
    d = y1.shape[1]
    pos = pos_ref[...]
    np2 = jnp.sum(pos * pos, axis=1, keepdims=True)
    padw = tab_ref.shape[1] - 4 - d
    tab_ref[...] = jnp.concatenate(
        [pos, np2, y1, jnp.zeros((r, padw), _F32)], axis=1)
    ysc_ref[...] = ysc
    _stats_update(st1_ref, y1, t)
    _stats_update(stsc_ref, ysc, t)


def _run_prep1(x, pos, p_fc0, p_m, p_s, tab_w):
    n = x.shape[0]
    tile = 256
    grid = (n // tile,)
    dm = p_m["W"].shape[1]
    ds = p_s["W"].shape[1]
    return pl.pallas_call(
        _prep1_body,
        grid=grid,
        in_specs=[
            pl.BlockSpec((tile, x.shape[1]), lambda t: (t, 0)),
            pl.BlockSpec((tile, 3), lambda t: (t, 0)),
            _full_spec(p_fc0["W"].shape), _full_spec((1, p_fc0["W"].shape[1])),
            _full_spec(p_m["W"].shape), _full_spec((1, dm)),
            _full_spec(p_s["W"].shape), _full_spec((1, ds)),
        ],
        out_specs=[
            pl.BlockSpec((tile, tab_w), lambda t: (t, 0)),
            pl.BlockSpec((tile, ds), lambda t: (t, 0)),
            pl.BlockSpec((8, dm), lambda t: (0, 0)),
            pl.BlockSpec((8, ds), lambda t: (0, 0)),
        ],
        out_shape=[
            jax.ShapeDtypeStruct((n, tab_w), _F32),
            jax.ShapeDtypeStruct((n, ds), _F32),
            jax.ShapeDtypeStruct((8, dm), _F32),
            jax.ShapeDtypeStruct((8, ds), _F32),
        ],
    )(x, pos, p_fc0["W"], p_fc0["b"].reshape(1, -1),
      p_m["W"], p_m["b"].reshape(1, -1),
      p_s["W"], p_s["b"].reshape(1, -1))


def _run_prep2(h, pos, p_m, p_s, tab_w):
    n = h.shape[0]
    tile = 256
    grid = (n // tile,)
    dm = p_m["W"].shape[1]
    ds = p_s["W"].shape[1]
    return pl.pallas_call(
        _prep2_body,
        grid=grid,
        in_specs=[
            pl.BlockSpec((tile, h.shape[1]), lambda t: (t, 0)),
            pl.BlockSpec((tile, 3), lambda t: (t, 0)),
            _full_spec(p_m["W"].shape), _full_spec((1, dm)),
            _full_spec(p_s["W"].shape), _full_spec((1, ds)),
        ],
        out_specs=[
            pl.BlockSpec((tile, tab_w), lambda t: (t, 0)),
            pl.BlockSpec((tile, ds), lambda t: (t, 0)),
            pl.BlockSpec((8, dm), lambda t: (0, 0)),
            pl.BlockSpec((8, ds), lambda t: (0, 0)),
        ],
        out_shape=[
            jax.ShapeDtypeStruct((n, tab_w), _F32),
            jax.ShapeDtypeStruct((n, ds), _F32),
            jax.ShapeDtypeStruct((8, dm), _F32),
            jax.ShapeDtypeStruct((8, ds), _F32),
        ],
    )(h, pos, p_m["W"], p_m["b"].reshape(1, -1),
      p_s["W"], p_s["b"].reshape(1, -1))


_KNN_ROWS = 128


def _tree(op, xs):
    while len(xs) > 1:
        nxt = [op(xs[i], xs[i + 1]) for i in range(0, len(xs) - 1, 2)]
        if len(xs) % 2:
            nxt.append(xs[-1])
        xs = nxt
    return xs[0]


def _knn_body(l, pos_ref, posT_ref, idx_ref):
    b = pl.program_id(0)
    tile = pos_ref[0]
    d = None
    for c in range(3):
        tc = tile[:, c:c + 1]
        fc = posT_ref[0, c:c + 1, :]
        dc = (tc - fc) ** 2
        d = dc if d is None else d + dc
    nb = l // 128
    ii = jax.lax.broadcasted_iota(jnp.int32, (_KNN_ROWS, 128), 1).astype(_F32)
    dk = []
    for g in range(nb):
        ds = jnp.maximum(d[:, g * 128:(g + 1) * 128], _F32(1e-30))
        bits = jax.lax.bitcast_convert_type(ds, jnp.int32)
        dk.append(jax.lax.bitcast_convert_type(
            (bits & jnp.int32(~15)) | jnp.int32(g), _F32))
    fold = _tree(jnp.minimum, list(dk))
    big = _F32(3.0e38)
    cols = []
    for it in range(KK):
        m = jnp.min(fold, axis=1, keepdims=True)
        jl = jnp.min(jnp.where(fold == m, ii, _F32(128.0)), axis=1,
                     keepdims=True)
        gv = jax.lax.bitcast_convert_type(m, jnp.int32) & jnp.int32(15)
        cols.append(gv * 128 + jl.astype(jnp.int32))
        if it + 1 < KK:
            fold = None
            for g in range(nb):
                dkg = jnp.where(dk[g] == m, big, dk[g])
                dk[g] = dkg
                fold = dkg if fold is None else jnp.minimum(fold, dkg)
    idx = jnp.concatenate(cols, axis=1)
    idx_ref[0] = idx + b * l


def _run_knn(pos, b, l):
    pos3 = pos.reshape(b, l, 3)
    posT = jnp.transpose(pos3, (0, 2, 1))
    grid = (b, l // _KNN_ROWS)
    return pl.pallas_call(
        functools.partial(_knn_body, l),
        grid=grid,
        in_specs=[
            pl.BlockSpec((1, _KNN_ROWS, 3), lambda bi, t: (bi, t, 0)),
            pl.BlockSpec((1, 3, l), lambda bi, t: (bi, 0, 0)),
        ],
        out_specs=pl.BlockSpec((1, _KNN_ROWS, KK), lambda bi, t: (bi, t, 0)),
        out_shape=jax.ShapeDtypeStruct((b, l, KK), jnp.int32),
    )(pos3, posT)


def _f32_to_i8(t):
    n, w = t.shape
    return jax.lax.bitcast_convert_type(t, jnp.int8).reshape(n, 4 * w)


def _i8_to_f32(t):
    n, w = t.shape
    return jax.lax.bitcast_convert_type(
        t.reshape(n, w // 4, 4), jnp.float32)


def _sc_gather(table, idx):
    e = idx.shape[0]
    w = table.shape[1]
    window = 128
    idx2 = idx.reshape(1, e)
    mesh = plsc.VectorSubcoreMesh(core_axis_name="core",
                                  subcore_axis_name="subcore")

    @functools.partial(
        pl.kernel,
        out_type=jax.ShapeDtypeStruct((e, w), table.dtype),
        mesh=mesh,
    )
    def _gather_kernel(x_hbm, i_hbm, o_hbm):
        def body(i_vmem, o_vmem):
            pltpu.sync_copy(x_hbm.at[i_vmem.at[0]], o_vmem)

        pltpu.emit_pipeline(
            body,
            grid=(e // window,),
            in_specs=[pl.BlockSpec((1, window), index_map=lambda i: (0, i))],
            out_specs=[pl.BlockSpec((window, w), index_map=lambda i: (i, 0))],
            core_axis_name=("core", "subcore"),
            dimension_semantics=(pltpu.PARALLEL,),
        )(i_hbm, o_hbm)

    return _gather_kernel(table, idx2)


def _fold_groups(x, d):
    w = x.shape[1]
    while w > d:
        w //= 2
        x = x[:, :w] + x[:, w:2 * w]
    return x


def _col_stats(st_ref, y, step, d):
    @pl.when(step == 0)
    def _():
        st_ref[...] = jnp.zeros_like(st_ref)

    r = y.shape[0]
    ones = jnp.ones((8, r), _F32)
    s1 = _fold_groups(_mm(ones, y)[0:1, :], d)
    s2 = _fold_groups(_mm(ones, y * y)[0:1, :], d)
    pad = jnp.zeros((6, d), _F32)
    st_ref[...] += jnp.concatenate([s1, s2, pad], axis=0)


def _bn_coeffs(st, n, gamma, beta):
    d = gamma.shape[-1]
    s1 = st[0:1, :d]
    s2 = st[1:2, :d]
    m = s1 / n
    v = s2 / n - m * m
    scale = gamma / jnp.sqrt(v + 1e-6)
    shift = beta - m * scale
    return scale, shift


_TABW = 128


def _np_sel_enc():
    s = np.zeros((KK * _TABW, 112), np.float32)
    for g in range(KK):
        for c in range(3):
            s[_TABW * g + c, 16 * c + g] = 1.0
            s[_TABW * g + c, 48 + 3 * g + c] = 1.0
        s[_TABW * g + 3, 96 + g] = 1.0
    return jnp.asarray(s)


def _np_tile_eye(d):
    s = np.zeros((d, KK * d), np.float32)
    for k in range(KK):
        for c in range(d):
            s[c, k * d + c] = 1.0
    return jnp.asarray(s)


def _enc_derived(p_enc):
    w = p_enc["W"]
    d = w.shape[1]
    bpart = w[3:6] + w[6:9]
    apart = w[0:3] - w[6:9]
    bk = jnp.kron(jnp.eye(KK, dtype=_F32), bpart)
    ta = jnp.tile(apart, (1, KK))
    dm = jnp.kron(jnp.eye(KK, dtype=_F32), w[9:10])
    bw = jnp.tile(p_enc["b"].reshape(1, -1), (1, KK))
    return bk, ta, dm, bw, d


def _enc_body(g_ref, pos_ref, sall, b1k, ta1, d1m, b1w, b2k, ta2, d2m, b2w,
              y1_ref, y2_ref, st1_ref, st2_ref):
    t = pl.program_id(0)
    sel = _mm(g_ref[...], sall[...])
    pos_i = pos_ref[...]
    dx = sel[:, 0:16] - pos_i[:, 0:1]
    dy = sel[:, 16:32] - pos_i[:, 1:2]
    dz = sel[:, 32:48] - pos_i[:, 2:3]
    dist = jnp.sqrt(dx * dx + dy * dy + dz * dz)
    pjint = sel[:, 48:96]
    y1 = _mm(pjint, b1k[...]) + _mm(pos_i, ta1[...]) + _mm(dist, d1m[...]) \
        + b1w[...]
    y2 = _mm(pjint, b2k[...]) + _mm(pos_i, ta2[...]) + _mm(dist, d2m[...]) \
        + b2w[...]
    y1_ref[...] = y1
    y2_ref[...] = y2
    d1 = st1_ref.shape[1]
    d2 = st2_ref.shape[1]
    _col_stats(st1_ref, y1, t, d1)
    _col_stats(st2_ref, y2, t, d2)


def _run_enc(gw, pos, p_enc1, p_enc2):
    n = gw.shape[0]
    rtile = 256
    grid = (n // rtile,)
    b1k, ta1, d1m, b1w, d1 = _enc_derived(p_enc1)
    b2k, ta2, d2m, b2w, d2 = _enc_derived(p_enc2)
    sall = _np_sel_enc()
    args = [gw, pos, sall, b1k, ta1, d1m, b1w, b2k, ta2, d2m, b2w]
    in_specs = [
        pl.BlockSpec((rtile, gw.shape[1]), lambda t: (t, 0)),
        pl.BlockSpec((rtile, 3), lambda t: (t, 0)),
    ] + [_full_spec(a.shape) for a in args[2:]]
    return pl.pallas_call(
        _enc_body,
        grid=grid,
        in_specs=in_specs,
        out_specs=[
            pl.BlockSpec((rtile, KK * d1), lambda t: (t, 0)),
            pl.BlockSpec((rtile, KK * d2), lambda t: (t, 0)),
            pl.BlockSpec((8, d1), lambda t: (0, 0)),
            pl.BlockSpec((8, d2), lambda t: (0, 0)),
        ],
        out_shape=[
            jax.ShapeDtypeStruct((n, KK * d1), _F32),
            jax.ShapeDtypeStruct((n, KK * d2), _F32),
            jax.ShapeDtypeStruct((8, d1), _F32),
            jax.ShapeDtypeStruct((8, d2), _F32),
        ],
    )(*args)


def _np_sel_x(c0, dh):
    s = np.zeros((KK * _TABW, KK * dh), np.float32)
    for g in range(KK):
        for c in range(dh):
            s[_TABW * g + c0 + c, g * dh + c] = 1.0
    return jnp.asarray(s)


def _np_place(din, off, d):
    s = np.zeros((KK * din, KK * d), np.float32)
    for k in range(KK):
        for c in range(din):
            s[k * din + c, k * d + off + c] = 1.0
    return jnp.asarray(s)


def _fold_pair(x, d, op):
    while x.shape[1] > d:
        w = x.shape[1] // 2
        x = op(x[:, :w], x[:, w:])
    return x


def _lfa_body(n_nodes, n_edges, dh, g_ref, stx_ref, gx, bx,
              ye_ref, ste_ref, ge, be, sxsel, wax, wae, px, pe, t2,
              wp, bp, out_ref, stp_ref):
    t = pl.program_id(0)
    de = ge.shape[-1]
    d = dh + de
    sx, tx = _bn_coeffs(stx_ref[...], n_nodes, gx[...], bx[...])
    se, te = _bn_coeffs(ste_ref[...], n_edges, ge[...], be[...])
    sxw = jnp.tile(sx, (1, KK))
    txw = jnp.tile(tx, (1, KK))
    sew = jnp.tile(se, (1, KK))
    tew = jnp.tile(te, (1, KK))
    xw = _lrelu(_mm(g_ref[...], sxsel[...]) * sxw + txw)
    lw = _lrelu(ye_ref[...] * sew + tew)
    att = _mm(xw, wax[...]) + _mm(lw, wae[...])
    m = _fold_pair(att, d, jnp.maximum)
    e = jnp.exp(att - _mm(m, t2[...]))
    s = _fold_pair(e, d, jnp.add)
    sc = e * _mm(1.0 / (s + 1e-16), t2[...])
    localw = _mm(xw, px[...]) + _mm(lw, pe[...])
    out = _fold_pair(sc * localw, d, jnp.add)
    yp = _mm(out, wp[...]) + bp[...]
    r, dp = yp.shape
    padw = out_ref.shape[1] - dp
    if padw:
        out_ref[...] = jnp.concatenate(
            [yp, jnp.zeros((r, padw), _F32)], axis=1)
    else:
        out_ref[...] = yp
    _col_stats(stp_ref, yp, t, dp)


def _run_lfa(gw, stx, p_x, yenc, stenc, p_enc, p_att, p_post,
             n_nodes, c0, dh, out_w):
    rtile = 256
    grid = (n_nodes // rtile,)
    de = p_enc["W"].shape[1]
    d = dh + de
    dp = p_post["W"].shape[1]
    wa = p_att["W"]
    eye = jnp.eye(KK, dtype=_F32)
    args = [
        gw, stx,
        p_x["gamma"].reshape(1, -1), p_x["beta"].reshape(1, -1),
        yenc, stenc,
        p_enc["gamma"].reshape(1, -1), p_enc["beta"].reshape(1, -1),
        _np_sel_x(c0, dh),
        jnp.kron(eye, wa[:dh, :]), jnp.kron(eye, wa[dh:, :]),
        _np_place(dh, 0, d), _np_place(de, dh, d), _np_tile_eye(d),
        p_post["W"], p_post["b"].reshape(1, -1),
    ]
    body = functools.partial(_lfa_body, float(n_nodes),
                             float(n_nodes * KK), dh)
    in_specs = [
        pl.BlockSpec((rtile, gw.shape[1]), lambda t: (t, 0)),
        _full_spec(stx.shape),
        _full_spec((1, dh)), _full_spec((1, dh)),
        pl.BlockSpec((rtile, KK * de), lambda t: (t, 0)),
        _full_spec((8, de)),
        _full_spec((1, de)), _full_spec((1, de)),
    ] + [_full_spec(a.shape) for a in args[8:]]
    return pl.pallas_call(
        body,
        grid=grid,
        in_specs=in_specs,
        out_specs=[
            pl.BlockSpec((rtile, out_w), lambda t: (t, 0)),
            pl.BlockSpec((8, dp), lambda t: (0, 0)),
        ],
        out_shape=[
            jax.ShapeDtypeStruct((n_nodes, out_w), _F32),
            jax.ShapeDtypeStruct((8, dp), _F32),
        ],
    )(*args)


def _bnlin_body(n, din, y_ref, st_ref, g, b, w, bb, yo_ref, sto_ref):
    t = pl.program_id(0)
    h = _bn_apply(y_ref[...][:, :din], st_ref[...], n, g[...], b[...],
                  act=True)
    y = _mm(h, w[...]) + bb[...]
    yo_ref[...] = y
    _stats_update(sto_ref, y, t)


def _run_bnlin(yin, stin, p_in, p_lin):
    n, w_in = yin.shape
    din = p_lin["W"].shape[0]
    dout = p_lin["W"].shape[1]
    tile = 256
    grid = (n // tile,)
    body = functools.partial(_bnlin_body, float(n), din)
    return pl.pallas_call(
        body,
        grid=grid,
        in_specs=[
            pl.BlockSpec((tile, w_in), lambda t: (t, 0)),
            _full_spec(stin.shape),
            _full_spec((1, din)), _full_spec((1, din)),
            _full_spec(p_lin["W"].shape), _full_spec((1, dout)),
        ],
        out_specs=[
            pl.BlockSpec((tile, dout), lambda t: (t, 0)),
            pl.BlockSpec((8, dout), lambda t: (0, 0)),
        ],
        out_shape=[
            jax.ShapeDtypeStruct((n, dout), _F32),
            jax.ShapeDtypeStruct((8, dout), _F32),
        ],
    )(yin, stin, p_in["gamma"].reshape(1, -1), p_in["beta"].reshape(1, -1),
      p_lin["W"], p_lin["b"].reshape(1, -1))


def _res_body(n, ym_ref, stm_ref, gm, bm, ysc_ref, stsc_ref, gs, bs, h_ref):
    a = _bn_apply(ym_ref[...], stm_ref[...], n, gm[...], bm[...], act=False)
    c = _bn_apply(ysc_ref[...], stsc_ref[...], n, gs[...], bs[...], act=False)
    h_ref[...] = _lrelu(a + c)


def _run_res(ym, stm, p_m, ysc, stsc, p_s):
    n, d = ym.shape
    tile = 256
    grid = (n // tile,)
    body = functools.partial(_res_body, float(n))
    return pl.pallas_call(
        body,
        grid=grid,
        in_specs=[
            pl.BlockSpec((tile, d), lambda t: (t, 0)),
            _full_spec(stm.shape),
            _full_spec((1, d)), _full_spec((1, d)),
            pl.BlockSpec((tile, d), lambda t: (t, 0)),
            _full_spec(stsc.shape),
            _full_spec((1, d)), _full_spec((1, d)),
        ],
        out_specs=pl.BlockSpec((tile, d), lambda t: (t, 0)),
        out_shape=jax.ShapeDtypeStruct((n, d), _F32),
    )(ym, stm, p_m["gamma"].reshape(1, -1), p_m["beta"].reshape(1, -1),
      ysc, stsc, p_s["gamma"].reshape(1, -1), p_s["beta"].reshape(1, -1))


def _head_body(h_ref, w1, b1, g1, be1, we1, be_1, ge1, bee1, we2, be_2,
               o_ref):
    h = h_ref[...]
    y = _mm(h, w1[...]) + b1[...]
    m = jnp.mean(y, axis=0)
    v = jnp.mean((y - m) ** 2, axis=0)
    y = (y - m) / jnp.sqrt(v + 1e-6) * g1[...] + be1[...]
    y = _lrelu(y)
    g = jnp.max(y.reshape(BB, -1, y.shape[-1]), axis=1)
    o = _mm(g, we1[...]) + be_1[...]
    m2 = jnp.mean(o, axis=0)
    v2 = jnp.mean((o - m2) ** 2, axis=0)
    o = (o - m2) / jnp.sqrt(v2 + 1e-6) * ge1[...] + bee1[...]
    o = _lrelu(o)
    logits = _mm(o, we2[...]) + be_2[...]
    lmax = jnp.max(logits, axis=-1, keepdims=True)
    s = logits - lmax
    lse = jnp.log(jnp.sum(jnp.exp(s), axis=-1, keepdims=True))
    o_ref[...] = s - lse


def _run_head(h, p1, pe1, pe2):
    return pl.pallas_call(
        _head_body,
        out_shape=jax.ShapeDtypeStruct((BB, NUM_CLASSES), _F32),
    )(h, p1["W"], p1["b"].reshape(1, -1), p1["gamma"].reshape(1, -1),
      p1["beta"].reshape(1, -1),
      pe1["W"], pe1["b"].reshape(1, -1), pe1["gamma"].reshape(1, -1),
      pe1["beta"].reshape(1, -1),
      pe2["W"], pe2["b"].reshape(1, -1))


def _block(p, prep_out, pos, b, l):
    n = b * l
    tab, ysc, st1, stsc = prep_out
    dm = st1.shape[1]

    idx = _run_knn(pos, b, l)
    idx_flat = idx.reshape(-1)

    g1 = _sc_gather(tab, idx_flat)
    g1w = g1.reshape(n, KK * _TABW)

    yenc1, yenc2, ste1, ste2 = _run_enc(g1w, pos, p["lfa1"]["enc"],
                                        p["lfa2"]["enc"])

    d2h = p["lfa2"]["enc"]["W"].shape[1]
    ypost1, stp1 = _run_lfa(
        g1w, st1, p["mlp1"], yenc1, ste1, p["lfa1"]["enc"],
        p["lfa1"]["att"], p["lfa1"]["post"],
        n_nodes=n, c0=4, dh=dm, out_w=_TABW)

    g2 = _sc_gather(ypost1, idx_flat)
    g2w = g2.reshape(n, KK * _TABW)

    ypost2, stp2 = _run_lfa(
        g2w, stp1, p["lfa1"]["post"], yenc2, ste2, p["lfa2"]["enc"],
        p["lfa2"]["att"], p["lfa2"]["post"],
        n_nodes=n, c0=0, dh=d2h, out_w=2 * d2h)

    ym2, stm2 = _run_bnlin(ypost2, stp2, p["lfa2"]["post"], p["mlp2"])

    h = _run_res(ym2, stm2, p["mlp2"], ysc, stsc, p["shortcut"])
    return h


def kernel(x, pos, batch, params):
    p1 = params["block1"]
    p2 = params["block2"]

    prep1 = _run_prep1(x, pos, params["fc0"], p1["mlp1"], p1["shortcut"],
                       tab_w=_TABW)
    h = _block(p1, prep1, pos, BB, LL)

    h_dec = h[::DEC]
    pos1 = pos[::DEC]

    prep2 = _run_prep2(h_dec, pos1, p2["mlp1"], p2["shortcut"], tab_w=_TABW)
    h2 = _block(p2, prep2, pos1, BB, LL // DEC)

    h_head = h2[::DEC]
    return _run_head(h_head, params["mlp1"], params["end1"], params["end2"])
